# baseline (speedup 1.0000x reference)
import functools

import jax
import jax.numpy as jnp
from jax import lax
from jax.experimental import pallas as pl
from jax.experimental.pallas import tpu as pltpu
from jax.experimental.pallas import tpu_sc as plsc

N = 10000
E = 320000
NEX = 100000
NC, NS, L = 2, 16, 16
NW = NC * NS

EPT = E // NW
KCH = 512
NCH = 20
TAIL = EPT - (NCH - 1) * KCH
PAD = KCH - TAIL
NBUF = 4
Z_SH_ROWS = 10240
DUMMY = N
ACC_ROWS = 10112
RPS = ACC_ROWS // NS

EX_T = NEX // NW
EX_CH = 196
EX_OUT = EX_CH * L


def _f32(*shape):
    return jax.ShapeDtypeStruct(shape, jnp.float32)


@functools.cache
def _mesh():
    return plsc.VectorSubcoreMesh(
        core_axis_name="c", subcore_axis_name="s",
        num_cores=NC, num_subcores=NS)


_SC_PARAMS = pltpu.CompilerParams(
    use_tc_tiling_on_sc=False, needs_layout_passes=False)


def _load_edge_chunks(eidx_hbm, row, wid, idxv, pad_hbm):
    base = wid * EPT
    for c in range(NCH - 1):
        pltpu.sync_copy(eidx_hbm.at[row, pl.ds(base + c * KCH, KCH)],
                        idxv.at[c, pl.ds(0, KCH)])
    pltpu.sync_copy(eidx_hbm.at[row, pl.ds(base + (NCH - 1) * KCH, TAIL)],
                    idxv.at[NCH - 1, pl.ds(0, TAIL)])
    pltpu.sync_copy(pad_hbm, idxv.at[NCH - 1, pl.ds(TAIL, PAD)])


@functools.cache
def _make_sc_degree():
    @functools.partial(
        pl.kernel,
        out_type=[_f32(ACC_ROWS, 16), _f32(ACC_ROWS, 16)],
        mesh=_mesh(),
        compiler_params=_SC_PARAMS,
        scratch_types=[
            pltpu.VMEM((NCH, KCH), jnp.int32),
            pltpu.VMEM((KCH, 16), jnp.float32),
            pltpu.VMEM_SHARED((ACC_ROWS, 16), jnp.float32),
            pltpu.SemaphoreType.DMA,
        ],
    )
    def k(eidx_hbm, ones_hbm, zeros_hbm, pad_hbm, p0_hbm, p1_hbm,
          dstv, onesv, acc, sem):
        cid = lax.axis_index("c")
        sid = lax.axis_index("s")
        wid = cid * NS + sid
        rs = pl.ds(sid * RPS, RPS)
        _load_edge_chunks(eidx_hbm, 1, wid, dstv, pad_hbm)
        pltpu.sync_copy(ones_hbm, onesv)
        pltpu.sync_copy(zeros_hbm.at[rs], acc.at[rs])
        plsc.subcore_barrier()

        hs = [pltpu.async_copy(onesv, acc.at[dstv.at[c]], sem, add=True)
              for c in range(NCH)]
        for h in hs:
            h.wait()

        plsc.subcore_barrier()

        @pl.when(cid == 0)
        def _():
            pltpu.sync_copy(acc.at[rs], p0_hbm.at[rs])

        @pl.when(cid == 1)
        def _():
            pltpu.sync_copy(acc.at[rs], p1_hbm.at[rs])

    return k


@functools.cache
def _make_sc_aggregate(trows):
    @functools.partial(
        pl.kernel,
        out_type=[_f32(ACC_ROWS, 16), _f32(ACC_ROWS, 16)],
        mesh=_mesh(),
        compiler_params=_SC_PARAMS,
        scratch_types=[
            pltpu.VMEM((NCH, KCH), jnp.int32),
            pltpu.VMEM((NCH, KCH), jnp.int32),
            [pltpu.VMEM((KCH, 16), jnp.float32)] * NBUF,
            pltpu.VMEM_SHARED((Z_SH_ROWS, 16), jnp.float32),
            pltpu.VMEM_SHARED((ACC_ROWS, 16), jnp.float32),
            [pltpu.SemaphoreType.DMA] * NBUF,
            [pltpu.SemaphoreType.DMA] * NBUF,
        ],
    )
    def k(eidx_hbm, z_hbm, zeros_hbm, pad_hbm, p0_hbm, p1_hbm,
          srcv, dstv, bufs, z_sh, acc, gsems, ssems):
        cid = lax.axis_index("c")
        sid = lax.axis_index("s")
        wid = cid * NS + sid
        rs = pl.ds(sid * RPS, RPS)
        _load_edge_chunks(eidx_hbm, 0, wid, srcv, pad_hbm)
        _load_edge_chunks(eidx_hbm, 1, wid, dstv, pad_hbm)
        pltpu.sync_copy(zeros_hbm.at[rs], acc.at[rs])

        if trows == Z_SH_ROWS:
            zs = pl.ds(sid * 640, 640)
            pltpu.sync_copy(z_hbm.at[zs], z_sh.at[zs])
        else:
            @pl.when(sid < NS - 1)
            def _():
                zs = pl.ds(sid * 640, 640)
                pltpu.sync_copy(z_hbm.at[zs], z_sh.at[zs])

            @pl.when(sid == NS - 1)
            def _():
                zs = pl.ds((NS - 1) * 640, trows - (NS - 1) * 640)
                pltpu.sync_copy(z_hbm.at[zs], z_sh.at[zs])

        plsc.subcore_barrier()

        def fire_gather(c):
            return pltpu.async_copy(
                z_sh.at[srcv.at[c]], bufs[c % NBUF], gsems[c % NBUF])

        def fire_scatter(c):
            return pltpu.async_copy(
                bufs[c % NBUF], acc.at[dstv.at[c]], ssems[c % NBUF],
                add=True)

        depth = 2
        gh = {c: fire_gather(c) for c in range(depth)}
        sh = {}
        for c in range(NCH):
            gh[c].wait()
            sh[c] = fire_scatter(c)
            nxt = c + depth
            if nxt < NCH:
                if nxt >= NBUF:
                    sh[nxt - NBUF].wait()
                    del sh[nxt - NBUF]
                gh[nxt] = fire_gather(nxt)
        for c in sorted(sh):
            sh[c].wait()

        plsc.subcore_barrier()

        @pl.when(cid == 0)
        def _():
            pltpu.sync_copy(acc.at[rs], p0_hbm.at[rs])

        @pl.when(cid == 1)
        def _():
            pltpu.sync_copy(acc.at[rs], p1_hbm.at[rs])

    return k


@functools.cache
def _make_sc_score():
    @functools.partial(
        pl.kernel,
        out_type=_f32(NW, EX_OUT),
        mesh=_mesh(),
        compiler_params=_SC_PARAMS,
        scratch_types=[
            pltpu.VMEM((RPS, 16), jnp.float32),
            pltpu.VMEM((RPS, 16), jnp.float32),
            pltpu.VMEM((RPS, 16), jnp.float32),
            pltpu.VMEM((RPS, 16), jnp.float32),
            pltpu.VMEM((RPS,), jnp.float32),
            pltpu.VMEM((RPS,), jnp.float32),
            pltpu.VMEM((16,), jnp.float32),
            pltpu.VMEM_SHARED((ACC_ROWS,), jnp.float32),
            pltpu.VMEM_SHARED((ACC_ROWS,), jnp.float32),
            pltpu.VMEM((N,), jnp.float32),
            pltpu.VMEM((N,), jnp.float32),
            pltpu.VMEM((EX_T, 2), jnp.int32),
            pltpu.VMEM((EX_OUT,), jnp.float32),
        ],
    )
    def k(q0_hbm, q1_hbm, t_hbm, y_hbm, cv_hbm, ex_hbm, out_hbm,
          q0v, q1v, tv, yv, siv, sjv, cvv,
          si_sh, sj_sh, sit, sjt, exv, outv):
        cid = lax.axis_index("c")
        sid = lax.axis_index("s")
        wid = cid * NS + sid
        rs = pl.ds(sid * RPS, RPS)
        pltpu.sync_copy(q0_hbm.at[rs], q0v)
        pltpu.sync_copy(q1_hbm.at[rs], q1v)
        pltpu.sync_copy(t_hbm.at[rs], tv)
        pltpu.sync_copy(y_hbm.at[rs], yv)
        pltpu.sync_copy(cv_hbm, cvv)
        pltpu.sync_copy(ex_hbm.at[pl.ds(wid * EX_T, EX_T)], exv)

        iot = lax.iota(jnp.int32, L)
        zer = jnp.zeros((L,), jnp.int32)
        one = zer + 1
        ci = plsc.load_gather(cvv, [zer])
        cj = plsc.load_gather(cvv, [one])

        starts = [i * L for i in range(RPS // L)] + [RPS - L]
        for st in starts:
            ridx = iot + st
            yq = plsc.load_gather(yv, [ridx, zer])
            a0 = plsc.load_gather(q0v, [ridx, zer])
            a1 = plsc.load_gather(q1v, [ridx, zer])
            tt = plsc.load_gather(tv, [ridx, zer])
            siv.at[pl.ds(st, L)][...] = yq * (a0 + a1 + tt) + ci
            b0 = plsc.load_gather(q0v, [ridx, one])
            b1 = plsc.load_gather(q1v, [ridx, one])
            bt = plsc.load_gather(tv, [ridx, one])
            sjv.at[pl.ds(st, L)][...] = yq * (b0 + b1 + bt) + cj

        pltpu.sync_copy(siv, si_sh.at[rs])
        pltpu.sync_copy(sjv, sj_sh.at[rs])
        plsc.subcore_barrier()
        pltpu.sync_copy(si_sh.at[pl.ds(0, N)], sit)
        pltpu.sync_copy(sj_sh.at[pl.ds(0, N)], sjt)

        @pl.loop(0, EX_CH - 1)
        def _(c):
            st = c * L
            ridx = iot + st
            i0 = plsc.load_gather(exv, [ridx, zer])
            i1 = plsc.load_gather(exv, [ridx, one])
            a = plsc.load_gather(sit, [i0])
            b = plsc.load_gather(sjt, [i1])
            outv.at[pl.ds(st, L)][...] = 1.0 / (1.0 + jnp.exp(-(a + b)))

        st = EX_T - L
        ridx = iot + st
        i0 = plsc.load_gather(exv, [ridx, zer])
        i1 = plsc.load_gather(exv, [ridx, one])
        a = plsc.load_gather(sit, [i0])
        b = plsc.load_gather(sjt, [i1])
        outv.at[pl.ds(st, L)][...] = 1.0 / (1.0 + jnp.exp(-(a + b)))

        pltpu.sync_copy(outv, out_hbm.at[wid])

    return k


def _tc1_body(x_ref, w1_ref, d0_ref, d1_ref, z1_ref, y_ref):
    y16 = lax.rsqrt(d0_ref[...] + d1_ref[...] + 1.0)
    xw = jnp.dot(x_ref[...], w1_ref[...], preferred_element_type=jnp.float32,
                 precision=lax.Precision.HIGHEST)
    z1_ref[...] = xw * y16[0:N, :]
    y_ref[...] = y16


def _tc1(x, w1, d0, d1):
    return pl.pallas_call(
        _tc1_body, out_shape=[_f32(N, 16), _f32(ACC_ROWS, 16)]
    )(x, w1, d0, d1)


def _tc2_body(y_ref, z1_ref, p0_ref, p1_ref, b1_ref, w2_ref, wrep_ref, t_ref):
    y16 = y_ref[0:N, :]
    h1 = jnp.maximum(y16 * (p0_ref[0:N, :] + p1_ref[0:N, :] + z1_ref[...])
                     + b1_ref[...], 0.0)
    xw2 = jnp.dot(h1, w2_ref[...], preferred_element_type=jnp.float32,
                  precision=lax.Precision.HIGHEST)
    z2 = xw2 * jnp.concatenate([y16, y16], axis=1)
    t_ref[0:N, :] = jnp.dot(z2, wrep_ref[...],
                            preferred_element_type=jnp.float32,
                            precision=lax.Precision.HIGHEST)
    t_ref[N:Z_SH_ROWS, :] = jnp.zeros((Z_SH_ROWS - N, 16), jnp.float32)


def _tc2(y16, z1, p0, p1, b1, w2, wrep):
    return pl.pallas_call(_tc2_body, out_shape=_f32(Z_SH_ROWS, 16))(
        y16, z1, p0, p1, b1, w2, wrep)


def kernel(x, edge_index, examples, W1, b1, W2, b2, Wfc, bfc):
    eidx = edge_index.astype(jnp.int32)
    ex = examples.astype(jnp.int32)

    ones16 = jnp.ones((KCH, 16), jnp.float32)
    zeros16 = jnp.zeros((ACC_ROWS, 16), jnp.float32)
    padv = jnp.full((PAD,), DUMMY, jnp.int32)

    wi = Wfc[:32, 0]
    wj = Wfc[32:, 0]
    wrep = jnp.tile(jnp.stack([wi, wj], axis=1), (1, 8))
    cvec = jnp.tile(
        jnp.stack([jnp.dot(b2, wi) + bfc[0], jnp.dot(b2, wj)]), (8,))

    d0, d1 = _make_sc_degree()(eidx, ones16, zeros16, padv)
    z1, y16 = _tc1(x, W1, d0, d1)
    p0, p1 = _make_sc_aggregate(N)(eidx, z1, zeros16, padv)
    t16 = _tc2(y16, z1, p0, p1, b1.reshape(1, 16), W2, wrep)
    q0, q1 = _make_sc_aggregate(Z_SH_ROWS)(eidx, t16, zeros16, padv)
    out2d = _make_sc_score()(q0, q1, t16, y16, cvec, ex)
    return out2d[:, :EX_T].reshape(-1)


# speedup vs baseline: 1.5285x; 1.5285x over previous
import functools

import jax
import jax.numpy as jnp
from jax import lax
from jax.experimental import pallas as pl
from jax.experimental.pallas import tpu as pltpu
from jax.experimental.pallas import tpu_sc as plsc

N = 10000
E = 320000
NEX = 100000
NC, NS, L = 2, 16, 16
NW = NC * NS

EPT = E // NW
KCH = 512
NCH = 20
TAIL = EPT - (NCH - 1) * KCH
PAD = KCH - TAIL
NBUF = 4
Z_SH_ROWS = 10240
DUMMY = N
ACC_ROWS = 10112
RPS = ACC_ROWS // NS

EX_T = 3200
EX_PAD = NW * EX_T


def _f32(*shape):
    return jax.ShapeDtypeStruct(shape, jnp.float32)


@functools.cache
def _mesh():
    return plsc.VectorSubcoreMesh(
        core_axis_name="c", subcore_axis_name="s",
        num_cores=NC, num_subcores=NS)


_SC_PARAMS = pltpu.CompilerParams(
    use_tc_tiling_on_sc=False, needs_layout_passes=False)


def _load_edge_chunks(eidx_hbm, row, wid, idxv, pad_hbm, sem):
    base = wid * EPT
    hs = [pltpu.async_copy(eidx_hbm.at[row, pl.ds(base + c * KCH, KCH)],
                           idxv.at[c, pl.ds(0, KCH)], sem)
          for c in range(NCH - 1)]
    hs.append(pltpu.async_copy(
        eidx_hbm.at[row, pl.ds(base + (NCH - 1) * KCH, TAIL)],
        idxv.at[NCH - 1, pl.ds(0, TAIL)], sem))
    hs.append(pltpu.async_copy(
        pad_hbm, idxv.at[NCH - 1, pl.ds(TAIL, PAD)], sem))
    for h in hs:
        h.wait()


@functools.cache
def _make_sc_degree():
    @functools.partial(
        pl.kernel,
        out_type=[_f32(ACC_ROWS, 16), _f32(ACC_ROWS, 16)],
        mesh=_mesh(),
        compiler_params=_SC_PARAMS,
        scratch_types=[
            pltpu.VMEM((NCH, KCH), jnp.int32),
            pltpu.VMEM((KCH, 16), jnp.float32),
            pltpu.VMEM_SHARED((ACC_ROWS, 16), jnp.float32),
            pltpu.SemaphoreType.DMA,
        ],
    )
    def k(eidx_hbm, ones_hbm, zeros_hbm, pad_hbm, p0_hbm, p1_hbm,
          dstv, onesv, acc, sem):
        cid = lax.axis_index("c")
        sid = lax.axis_index("s")
        wid = cid * NS + sid
        rs = pl.ds(sid * RPS, RPS)
        _load_edge_chunks(eidx_hbm, 1, wid, dstv, pad_hbm, sem)
        pltpu.sync_copy(ones_hbm, onesv)
        pltpu.sync_copy(zeros_hbm.at[rs], acc.at[rs])
        plsc.subcore_barrier()

        hs = [pltpu.async_copy(onesv, acc.at[dstv.at[c]], sem, add=True)
              for c in range(NCH)]
        for h in hs:
            h.wait()

        plsc.subcore_barrier()

        @pl.when(cid == 0)
        def _():
            pltpu.sync_copy(acc.at[rs], p0_hbm.at[rs])

        @pl.when(cid == 1)
        def _():
            pltpu.sync_copy(acc.at[rs], p1_hbm.at[rs])

    return k


@functools.cache
def _make_sc_aggregate(trows):
    @functools.partial(
        pl.kernel,
        out_type=[_f32(ACC_ROWS, 16), _f32(ACC_ROWS, 16)],
        mesh=_mesh(),
        compiler_params=_SC_PARAMS,
        scratch_types=[
            pltpu.VMEM((NCH, KCH), jnp.int32),
            pltpu.VMEM((NCH, KCH), jnp.int32),
            [pltpu.VMEM((KCH, 16), jnp.float32)] * NBUF,
            pltpu.VMEM_SHARED((Z_SH_ROWS, 16), jnp.float32),
            pltpu.VMEM_SHARED((ACC_ROWS, 16), jnp.float32),
            [pltpu.SemaphoreType.DMA] * NBUF,
            [pltpu.SemaphoreType.DMA] * NBUF,
        ],
    )
    def k(eidx_hbm, z_hbm, zeros_hbm, pad_hbm, p0_hbm, p1_hbm,
          srcv, dstv, bufs, z_sh, acc, gsems, ssems):
        cid = lax.axis_index("c")
        sid = lax.axis_index("s")
        wid = cid * NS + sid
        rs = pl.ds(sid * RPS, RPS)
        _load_edge_chunks(eidx_hbm, 0, wid, srcv, pad_hbm, gsems[0])
        _load_edge_chunks(eidx_hbm, 1, wid, dstv, pad_hbm, gsems[1])
        pltpu.sync_copy(zeros_hbm.at[rs], acc.at[rs])

        if trows == Z_SH_ROWS:
            zs = pl.ds(sid * 640, 640)
            pltpu.sync_copy(z_hbm.at[zs], z_sh.at[zs])
        else:
            @pl.when(sid < NS - 1)
            def _():
                zs = pl.ds(sid * 640, 640)
                pltpu.sync_copy(z_hbm.at[zs], z_sh.at[zs])

            @pl.when(sid == NS - 1)
            def _():
                zs = pl.ds((NS - 1) * 640, trows - (NS - 1) * 640)
                pltpu.sync_copy(z_hbm.at[zs], z_sh.at[zs])

        plsc.subcore_barrier()

        def fire_gather(c):
            return pltpu.async_copy(
                z_sh.at[srcv.at[c]], bufs[c % NBUF], gsems[c % NBUF])

        def fire_scatter(c):
            return pltpu.async_copy(
                bufs[c % NBUF], acc.at[dstv.at[c]], ssems[c % NBUF],
                add=True)

        depth = 2
        gh = {c: fire_gather(c) for c in range(depth)}
        sh = {}
        for c in range(NCH):
            gh[c].wait()
            sh[c] = fire_scatter(c)
            nxt = c + depth
            if nxt < NCH:
                if nxt >= NBUF:
                    sh[nxt - NBUF].wait()
                    del sh[nxt - NBUF]
                gh[nxt] = fire_gather(nxt)
        for c in sorted(sh):
            sh[c].wait()

        plsc.subcore_barrier()

        @pl.when(cid == 0)
        def _():
            pltpu.sync_copy(acc.at[rs], p0_hbm.at[rs])

        @pl.when(cid == 1)
        def _():
            pltpu.sync_copy(acc.at[rs], p1_hbm.at[rs])

    return k


@functools.cache
def _make_sc_score():
    @functools.partial(
        pl.kernel,
        out_type=_f32(EX_PAD),
        mesh=_mesh(),
        compiler_params=_SC_PARAMS,
        scratch_types=[
            pltpu.VMEM((RPS, 16), jnp.float32),
            pltpu.VMEM((RPS, 16), jnp.float32),
            pltpu.VMEM((RPS, 16), jnp.float32),
            pltpu.VMEM((RPS, 16), jnp.float32),
            pltpu.VMEM((RPS,), jnp.float32),
            pltpu.VMEM((RPS,), jnp.float32),
            pltpu.VMEM((16,), jnp.float32),
            pltpu.VMEM_SHARED((ACC_ROWS,), jnp.float32),
            pltpu.VMEM_SHARED((ACC_ROWS,), jnp.float32),
            pltpu.VMEM((N,), jnp.float32),
            pltpu.VMEM((N,), jnp.float32),
            pltpu.VMEM((EX_T,), jnp.int32),
            pltpu.VMEM((EX_T,), jnp.int32),
            pltpu.VMEM((EX_T,), jnp.float32),
        ],
    )
    def k(q0_hbm, q1_hbm, t_hbm, y_hbm, cv_hbm, ex0_hbm, ex1_hbm, out_hbm,
          q0v, q1v, tv, yv, siv, sjv, cvv,
          si_sh, sj_sh, sit, sjt, e0v, e1v, outv):
        cid = lax.axis_index("c")
        sid = lax.axis_index("s")
        wid = cid * NS + sid
        rs = pl.ds(sid * RPS, RPS)
        pltpu.sync_copy(q0_hbm.at[rs], q0v)
        pltpu.sync_copy(q1_hbm.at[rs], q1v)
        pltpu.sync_copy(t_hbm.at[rs], tv)
        pltpu.sync_copy(y_hbm.at[rs], yv)
        pltpu.sync_copy(cv_hbm, cvv)
        pltpu.sync_copy(ex0_hbm.at[wid], e0v)
        pltpu.sync_copy(ex1_hbm.at[wid], e1v)

        iot = lax.iota(jnp.int32, L)
        zer = jnp.zeros((L,), jnp.int32)
        one = zer + 1
        ci = plsc.load_gather(cvv, [zer])
        cj = plsc.load_gather(cvv, [one])

        starts = [i * L for i in range(RPS // L)] + [RPS - L]
        for st in starts:
            ridx = iot + st
            yq = plsc.load_gather(yv, [ridx, zer])
            a0 = plsc.load_gather(q0v, [ridx, zer])
            a1 = plsc.load_gather(q1v, [ridx, zer])
            tt = plsc.load_gather(tv, [ridx, zer])
            siv.at[pl.ds(st, L)][...] = yq * (a0 + a1 + tt) + ci
            b0 = plsc.load_gather(q0v, [ridx, one])
            b1 = plsc.load_gather(q1v, [ridx, one])
            bt = plsc.load_gather(tv, [ridx, one])
            sjv.at[pl.ds(st, L)][...] = yq * (b0 + b1 + bt) + cj

        pltpu.sync_copy(siv, si_sh.at[rs])
        pltpu.sync_copy(sjv, sj_sh.at[rs])
        plsc.subcore_barrier()
        pltpu.sync_copy(si_sh.at[pl.ds(0, N)], sit)
        pltpu.sync_copy(sj_sh.at[pl.ds(0, N)], sjt)

        @pl.loop(0, EX_T, step=L)
        def _(i):
            i0 = e0v.at[pl.ds(i, L)][...]
            i1 = e1v.at[pl.ds(i, L)][...]
            a = plsc.load_gather(sit, [i0])
            b = plsc.load_gather(sjt, [i1])
            outv.at[pl.ds(i, L)][...] = 1.0 / (1.0 + jnp.exp(-(a + b)))

        pltpu.sync_copy(outv, out_hbm.at[pl.ds(wid * EX_T, EX_T)])

    return k


def _tc1_body(x_ref, w1_ref, d0_ref, d1_ref, z1_ref, y_ref):
    y16 = lax.rsqrt(d0_ref[...] + d1_ref[...] + 1.0)
    xw = jnp.dot(x_ref[...], w1_ref[...], preferred_element_type=jnp.float32,
                 precision=lax.Precision.HIGHEST)
    z1_ref[...] = xw * y16[0:N, :]
    y_ref[...] = y16


def _tc1(x, w1, d0, d1):
    return pl.pallas_call(
        _tc1_body, out_shape=[_f32(N, 16), _f32(ACC_ROWS, 16)]
    )(x, w1, d0, d1)


def _tc2_body(y_ref, z1_ref, p0_ref, p1_ref, b1_ref, w2_ref, wrep_ref, t_ref):
    y16 = y_ref[0:N, :]
    h1 = jnp.maximum(y16 * (p0_ref[0:N, :] + p1_ref[0:N, :] + z1_ref[...])
                     + b1_ref[...], 0.0)
    xw2 = jnp.dot(h1, w2_ref[...], preferred_element_type=jnp.float32,
                  precision=lax.Precision.HIGHEST)
    z2 = xw2 * jnp.concatenate([y16, y16], axis=1)
    t_ref[0:N, :] = jnp.dot(z2, wrep_ref[...],
                            preferred_element_type=jnp.float32,
                            precision=lax.Precision.HIGHEST)
    t_ref[N:Z_SH_ROWS, :] = jnp.zeros((Z_SH_ROWS - N, 16), jnp.float32)


def _tc2(y16, z1, p0, p1, b1, w2, wrep):
    return pl.pallas_call(_tc2_body, out_shape=_f32(Z_SH_ROWS, 16))(
        y16, z1, p0, p1, b1, w2, wrep)


def kernel(x, edge_index, examples, W1, b1, W2, b2, Wfc, bfc):
    eidx = edge_index.astype(jnp.int32)
    xpad = EX_PAD - NEX
    ex0 = jnp.concatenate(
        [examples[:, 0].astype(jnp.int32), jnp.zeros((xpad,), jnp.int32)]
    ).reshape(NW, EX_T)
    ex1 = jnp.concatenate(
        [examples[:, 1].astype(jnp.int32), jnp.zeros((xpad,), jnp.int32)]
    ).reshape(NW, EX_T)

    ones16 = jnp.ones((KCH, 16), jnp.float32)
    zeros16 = jnp.zeros((ACC_ROWS, 16), jnp.float32)
    padv = jnp.full((PAD,), DUMMY, jnp.int32)

    wi = Wfc[:32, 0]
    wj = Wfc[32:, 0]
    wrep = jnp.tile(jnp.stack([wi, wj], axis=1), (1, 8))
    cvec = jnp.tile(
        jnp.stack([jnp.dot(b2, wi) + bfc[0], jnp.dot(b2, wj)]), (8,))

    d0, d1 = _make_sc_degree()(eidx, ones16, zeros16, padv)
    z1, y16 = _tc1(x, W1, d0, d1)
    p0, p1 = _make_sc_aggregate(N)(eidx, z1, zeros16, padv)
    t16 = _tc2(y16, z1, p0, p1, b1.reshape(1, 16), W2, wrep)
    q0, q1 = _make_sc_aggregate(Z_SH_ROWS)(eidx, t16, zeros16, padv)
    out = _make_sc_score()(q0, q1, t16, y16, cvec, ex0, ex1)
    return out[:NEX]


# speedup vs baseline: 2.0431x; 1.3367x over previous
import functools

import jax
import jax.numpy as jnp
from jax import lax
from jax.experimental import pallas as pl
from jax.experimental.pallas import tpu as pltpu
from jax.experimental.pallas import tpu_sc as plsc

N = 10000
E = 320000
NEX = 100000
NC, NS, L = 2, 16, 16
NW = NC * NS

EPT = E // NW
KCH = 512
NCH = 20
TAIL = EPT - (NCH - 1) * KCH
PAD = KCH - TAIL
NBUF = 4
Z_SH_ROWS = 10240
DUMMY = N
ACC_ROWS = 10112
RPS = ACC_ROWS // NS

EX_T = 3200
EX_PAD = NW * EX_T


def _f32(*shape):
    return jax.ShapeDtypeStruct(shape, jnp.float32)


@functools.cache
def _mesh():
    return plsc.VectorSubcoreMesh(
        core_axis_name="c", subcore_axis_name="s",
        num_cores=NC, num_subcores=NS)


_SC_PARAMS = pltpu.CompilerParams(
    use_tc_tiling_on_sc=False, needs_layout_passes=False)


def _load_edge_chunks(eidx_hbm, row, wid, idxv, pad_hbm, sem):
    base = wid * EPT
    hs = [pltpu.async_copy(eidx_hbm.at[row, pl.ds(base + c * KCH, KCH)],
                           idxv.at[c, pl.ds(0, KCH)], sem)
          for c in range(NCH - 1)]
    hs.append(pltpu.async_copy(
        eidx_hbm.at[row, pl.ds(base + (NCH - 1) * KCH, TAIL)],
        idxv.at[NCH - 1, pl.ds(0, TAIL)], sem))
    hs.append(pltpu.async_copy(
        pad_hbm, idxv.at[NCH - 1, pl.ds(TAIL, PAD)], sem))
    for h in hs:
        h.wait()


@functools.cache
def _make_sc_degree():
    @functools.partial(
        pl.kernel,
        out_type=[_f32(ACC_ROWS, 16), _f32(ACC_ROWS, 16)],
        mesh=_mesh(),
        compiler_params=_SC_PARAMS,
        scratch_types=[
            pltpu.VMEM((NCH, KCH), jnp.int32),
            pltpu.VMEM((KCH, 16), jnp.float32),
            pltpu.VMEM_SHARED((ACC_ROWS, 16), jnp.float32),
            pltpu.SemaphoreType.DMA,
        ],
    )
    def k(eidx_hbm, ones_hbm, zeros_hbm, pad_hbm, p0_hbm, p1_hbm,
          dstv, onesv, acc, sem):
        cid = lax.axis_index("c")
        sid = lax.axis_index("s")
        wid = cid * NS + sid
        rs = pl.ds(sid * RPS, RPS)
        _load_edge_chunks(eidx_hbm, 1, wid, dstv, pad_hbm, sem)
        pltpu.sync_copy(ones_hbm, onesv)
        pltpu.sync_copy(zeros_hbm.at[rs], acc.at[rs])
        plsc.subcore_barrier()

        hs = [pltpu.async_copy(onesv, acc.at[dstv.at[c]], sem, add=True)
              for c in range(NCH)]
        for h in hs:
            h.wait()

        plsc.subcore_barrier()

        @pl.when(cid == 0)
        def _():
            pltpu.sync_copy(acc.at[rs], p0_hbm.at[rs])

        @pl.when(cid == 1)
        def _():
            pltpu.sync_copy(acc.at[rs], p1_hbm.at[rs])

    return k


@functools.cache
def _make_sc_aggregate(trows):
    @functools.partial(
        pl.kernel,
        out_type=[_f32(ACC_ROWS, 16), _f32(ACC_ROWS, 16)],
        mesh=_mesh(),
        compiler_params=_SC_PARAMS,
        scratch_types=[
            pltpu.VMEM((NCH, KCH), jnp.int32),
            pltpu.VMEM((NCH, KCH), jnp.int32),
            [pltpu.VMEM((KCH, 16), jnp.float32)] * NBUF,
            pltpu.VMEM_SHARED((Z_SH_ROWS, 16), jnp.float32),
            pltpu.VMEM_SHARED((ACC_ROWS, 16), jnp.float32),
            [pltpu.SemaphoreType.DMA] * NBUF,
            [pltpu.SemaphoreType.DMA] * NBUF,
        ],
    )
    def k(eidx_hbm, z_hbm, zeros_hbm, pad_hbm, p0_hbm, p1_hbm,
          srcv, dstv, bufs, z_sh, acc, gsems, ssems):
        cid = lax.axis_index("c")
        sid = lax.axis_index("s")
        wid = cid * NS + sid
        rs = pl.ds(sid * RPS, RPS)
        _load_edge_chunks(eidx_hbm, 0, wid, srcv, pad_hbm, gsems[0])
        _load_edge_chunks(eidx_hbm, 1, wid, dstv, pad_hbm, gsems[1])
        pltpu.sync_copy(zeros_hbm.at[rs], acc.at[rs])

        if trows == Z_SH_ROWS:
            zs = pl.ds(sid * 640, 640)
            pltpu.sync_copy(z_hbm.at[zs], z_sh.at[zs])
        else:
            @pl.when(sid < NS - 1)
            def _():
                zs = pl.ds(sid * 640, 640)
                pltpu.sync_copy(z_hbm.at[zs], z_sh.at[zs])

            @pl.when(sid == NS - 1)
            def _():
                zs = pl.ds((NS - 1) * 640, trows - (NS - 1) * 640)
                pltpu.sync_copy(z_hbm.at[zs], z_sh.at[zs])

        plsc.subcore_barrier()

        def fire_gather(c):
            return pltpu.async_copy(
                z_sh.at[srcv.at[c]], bufs[c % NBUF], gsems[c % NBUF])

        def fire_scatter(c):
            return pltpu.async_copy(
                bufs[c % NBUF], acc.at[dstv.at[c]], ssems[c % NBUF],
                add=True)

        depth = 2
        gh = {c: fire_gather(c) for c in range(depth)}
        sh = {}
        for c in range(NCH):
            gh[c].wait()
            sh[c] = fire_scatter(c)
            nxt = c + depth
            if nxt < NCH:
                if nxt >= NBUF:
                    sh[nxt - NBUF].wait()
                    del sh[nxt - NBUF]
                gh[nxt] = fire_gather(nxt)
        for c in sorted(sh):
            sh[c].wait()

        plsc.subcore_barrier()

        @pl.when(cid == 0)
        def _():
            pltpu.sync_copy(acc.at[rs], p0_hbm.at[rs])

        @pl.when(cid == 1)
        def _():
            pltpu.sync_copy(acc.at[rs], p1_hbm.at[rs])

    return k


@functools.cache
def _make_sc_score():
    @functools.partial(
        pl.kernel,
        out_type=_f32(EX_PAD),
        mesh=_mesh(),
        compiler_params=_SC_PARAMS,
        scratch_types=[
            pltpu.VMEM((RPS, 16), jnp.float32),
            pltpu.VMEM((RPS, 16), jnp.float32),
            pltpu.VMEM((RPS, 16), jnp.float32),
            pltpu.VMEM((RPS, 16), jnp.float32),
            pltpu.VMEM((RPS,), jnp.float32),
            pltpu.VMEM((RPS,), jnp.float32),
            pltpu.VMEM((16,), jnp.float32),
            pltpu.VMEM_SHARED((ACC_ROWS,), jnp.float32),
            pltpu.VMEM_SHARED((ACC_ROWS,), jnp.float32),
            pltpu.VMEM((N,), jnp.float32),
            pltpu.VMEM((N,), jnp.float32),
            pltpu.VMEM((EX_T,), jnp.int32),
            pltpu.VMEM((EX_T,), jnp.int32),
            pltpu.VMEM((EX_T,), jnp.float32),
        ],
    )
    def k(q0_hbm, q1_hbm, t_hbm, y_hbm, cv_hbm, ex0_hbm, ex1_hbm, out_hbm,
          q0v, q1v, tv, yv, siv, sjv, cvv,
          si_sh, sj_sh, sit, sjt, e0v, e1v, outv):
        cid = lax.axis_index("c")
        sid = lax.axis_index("s")
        wid = cid * NS + sid
        rs = pl.ds(sid * RPS, RPS)
        pltpu.sync_copy(q0_hbm.at[rs], q0v)
        pltpu.sync_copy(q1_hbm.at[rs], q1v)
        pltpu.sync_copy(t_hbm.at[rs], tv)
        pltpu.sync_copy(y_hbm.at[rs], yv)
        pltpu.sync_copy(cv_hbm, cvv)
        pltpu.sync_copy(ex0_hbm.at[wid], e0v)
        pltpu.sync_copy(ex1_hbm.at[wid], e1v)

        iot = lax.iota(jnp.int32, L)
        zer = jnp.zeros((L,), jnp.int32)
        one = zer + 1
        ci = plsc.load_gather(cvv, [zer])
        cj = plsc.load_gather(cvv, [one])

        starts = [i * L for i in range(RPS // L)] + [RPS - L]
        for st in starts:
            ridx = iot + st
            yq = plsc.load_gather(yv, [ridx, zer])
            a0 = plsc.load_gather(q0v, [ridx, zer])
            a1 = plsc.load_gather(q1v, [ridx, zer])
            tt = plsc.load_gather(tv, [ridx, zer])
            siv.at[pl.ds(st, L)][...] = yq * (a0 + a1 + tt) + ci
            b0 = plsc.load_gather(q0v, [ridx, one])
            b1 = plsc.load_gather(q1v, [ridx, one])
            bt = plsc.load_gather(tv, [ridx, one])
            sjv.at[pl.ds(st, L)][...] = yq * (b0 + b1 + bt) + cj

        pltpu.sync_copy(siv, si_sh.at[rs])
        pltpu.sync_copy(sjv, sj_sh.at[rs])
        plsc.subcore_barrier()
        pltpu.sync_copy(si_sh.at[pl.ds(0, N)], sit)
        pltpu.sync_copy(sj_sh.at[pl.ds(0, N)], sjt)

        @pl.loop(0, EX_T, step=L)
        def _(i):
            i0 = e0v.at[pl.ds(i, L)][...]
            i1 = e1v.at[pl.ds(i, L)][...]
            a = plsc.load_gather(sit, [i0])
            b = plsc.load_gather(sjt, [i1])
            outv.at[pl.ds(i, L)][...] = 1.0 / (1.0 + jnp.exp(-(a + b)))

        pltpu.sync_copy(outv, out_hbm.at[pl.ds(wid * EX_T, EX_T)])

    return k


NP1 = N // 8
NPA = ACC_ROWS // 8
NPT = Z_SH_ROWS // 8


def _tc1_body(x_ref, w1_ref, d0_ref, d1_ref, z1_ref, y_ref):
    yp = lax.rsqrt(d0_ref[...] + d1_ref[...] + 1.0)
    xw = jnp.dot(x_ref[...], w1_ref[...], preferred_element_type=jnp.float32,
                 precision=lax.Precision.HIGHEST)
    z1_ref[...] = xw * yp[0:NP1, :]
    y_ref[...] = yp


def _tc1(xp, w1blk, d0p, d1p):
    return pl.pallas_call(
        _tc1_body, out_shape=[_f32(NP1, 128), _f32(NPA, 128)]
    )(xp, w1blk, d0p, d1p)


def _tc2_body(y_ref, z1_ref, p0_ref, p1_ref, b1_ref, w2_ref, wrep_ref, t_ref):
    yp = y_ref[0:NP1, :]
    h1 = jnp.maximum(yp * (p0_ref[0:NP1, :] + p1_ref[0:NP1, :] + z1_ref[...])
                     + b1_ref[...], 0.0)
    xw2 = jnp.dot(h1, w2_ref[...], preferred_element_type=jnp.float32,
                  precision=lax.Precision.HIGHEST)
    t_ref[0:NP1, :] = jnp.dot(xw2, wrep_ref[...],
                              preferred_element_type=jnp.float32,
                              precision=lax.Precision.HIGHEST) * yp
    t_ref[NP1:NPT, :] = jnp.zeros((NPT - NP1, 128), jnp.float32)


def _tc2(yp, z1p, p0p, p1p, b1rep, w2blk, wrepblk):
    return pl.pallas_call(_tc2_body, out_shape=_f32(NPT, 128))(
        yp, z1p, p0p, p1p, b1rep, w2blk, wrepblk)


def kernel(x, edge_index, examples, W1, b1, W2, b2, Wfc, bfc):
    eidx = edge_index.astype(jnp.int32)
    xpad = EX_PAD - NEX
    ex0 = jnp.concatenate(
        [examples[:, 0].astype(jnp.int32), jnp.zeros((xpad,), jnp.int32)]
    ).reshape(NW, EX_T)
    ex1 = jnp.concatenate(
        [examples[:, 1].astype(jnp.int32), jnp.zeros((xpad,), jnp.int32)]
    ).reshape(NW, EX_T)

    ones16 = jnp.ones((KCH, 16), jnp.float32)
    zeros16 = jnp.zeros((ACC_ROWS, 16), jnp.float32)
    padv = jnp.full((PAD,), DUMMY, jnp.int32)

    wi = Wfc[:32, 0]
    wj = Wfc[32:, 0]
    wrep = jnp.tile(jnp.stack([wi, wj], axis=1), (1, 8))
    cvec = jnp.tile(
        jnp.stack([jnp.dot(b2, wi) + bfc[0], jnp.dot(b2, wj)]), (8,))
    eye8 = jnp.eye(8, dtype=jnp.float32)
    w1blk = jnp.kron(eye8, W1)
    w2blk = jnp.kron(eye8, W2)
    wrepblk = jnp.kron(eye8, wrep)
    b1rep = jnp.tile(b1.reshape(1, 16), (1, 8))
    xp = x.reshape(NP1, 1024)

    d0, d1 = _make_sc_degree()(eidx, ones16, zeros16, padv)
    z1p, yp = _tc1(xp, w1blk, d0.reshape(NPA, 128), d1.reshape(NPA, 128))
    p0, p1 = _make_sc_aggregate(N)(eidx, z1p.reshape(N, 16), zeros16, padv)
    tp = _tc2(yp, z1p, p0.reshape(NPA, 128), p1.reshape(NPA, 128),
              b1rep, w2blk, wrepblk)
    t16 = tp.reshape(Z_SH_ROWS, 16)
    q0, q1 = _make_sc_aggregate(Z_SH_ROWS)(eidx, t16, zeros16, padv)
    out = _make_sc_score()(q0, q1, t16, yp.reshape(ACC_ROWS, 16),
                           cvec, ex0, ex1)
    return out[:NEX]


# speedup vs baseline: 2.1463x; 1.0505x over previous
import functools

import jax
import jax.numpy as jnp
from jax import lax
from jax.experimental import pallas as pl
from jax.experimental.pallas import tpu as pltpu
from jax.experimental.pallas import tpu_sc as plsc

N = 10000
E = 320000
NEX = 100000
NC, NS, L = 2, 16, 16
NW = NC * NS

EPT = E // NW
KCH = 512
NCH = 20
TAIL = EPT - (NCH - 1) * KCH
PAD = KCH - TAIL
NBUF = 4
Z_SH_ROWS = 10240
DUMMY = N
ACC_ROWS = 10112
RPS = ACC_ROWS // NS

EX_T = 3200
EX_PAD = NW * EX_T


def _f32(*shape):
    return jax.ShapeDtypeStruct(shape, jnp.float32)


@functools.cache
def _mesh():
    return plsc.VectorSubcoreMesh(
        core_axis_name="c", subcore_axis_name="s",
        num_cores=NC, num_subcores=NS)


_SC_PARAMS = pltpu.CompilerParams(
    use_tc_tiling_on_sc=False, needs_layout_passes=False)


def _load_edge_chunks(eidx_hbm, row, wid, idxv, pad_hbm, sem):
    base = wid * EPT
    hs = [pltpu.async_copy(eidx_hbm.at[row, pl.ds(base + c * KCH, KCH)],
                           idxv.at[c, pl.ds(0, KCH)], sem)
          for c in range(NCH - 1)]
    hs.append(pltpu.async_copy(
        eidx_hbm.at[row, pl.ds(base + (NCH - 1) * KCH, TAIL)],
        idxv.at[NCH - 1, pl.ds(0, TAIL)], sem))
    hs.append(pltpu.async_copy(
        pad_hbm, idxv.at[NCH - 1, pl.ds(TAIL, PAD)], sem))
    for h in hs:
        h.wait()


@functools.cache
def _make_sc_degree():
    @functools.partial(
        pl.kernel,
        out_type=[_f32(ACC_ROWS * 16), _f32(ACC_ROWS * 16)],
        mesh=_mesh(),
        compiler_params=_SC_PARAMS,
        scratch_types=[
            pltpu.VMEM((EPT,), jnp.int32),
            pltpu.VMEM((ACC_ROWS,), jnp.float32),
            pltpu.VMEM_SHARED((NS, ACC_ROWS), jnp.float32),
            pltpu.VMEM((NS * RPS,), jnp.float32),
            pltpu.VMEM((RPS,), jnp.float32),
            pltpu.VMEM((RPS * 16,), jnp.float32),
            pltpu.SemaphoreType.DMA,
        ],
    )
    def k(eidx_hbm, p0_hbm, p1_hbm, dstv, tbl, slab_sh, slabv, sumv, repv,
          sem):
        cid = lax.axis_index("c")
        sid = lax.axis_index("s")
        wid = cid * NS + sid
        pltpu.sync_copy(eidx_hbm.at[1, pl.ds(wid * EPT, EPT)], dstv)

        zf = jnp.zeros((L,), jnp.float32)
        onef = zf + 1.0

        @pl.loop(0, ACC_ROWS, step=L)
        def _(i):
            tbl.at[pl.ds(i, L)][...] = zf

        @pl.loop(0, EPT, step=L)
        def _(e):
            idx = dstv.at[pl.ds(e, L)][...]
            plsc.addupdate_scatter(tbl, [idx], onef)

        pltpu.sync_copy(tbl, slab_sh.at[sid])
        plsc.subcore_barrier()

        hs = [pltpu.async_copy(slab_sh.at[r, pl.ds(sid * RPS, RPS)],
                               slabv.at[pl.ds(r * RPS, RPS)], sem)
              for r in range(NS)]
        for h in hs:
            h.wait()

        starts = [i * L for i in range(RPS // L)] + [RPS - L]
        for st in starts:
            tot = slabv.at[pl.ds(st, L)][...]
            for r in range(1, NS):
                tot = tot + slabv.at[pl.ds(r * RPS + st, L)][...]
            sumv.at[pl.ds(st, L)][...] = tot

        zi = jnp.zeros((L,), jnp.int32)

        @pl.loop(0, RPS, step=4)
        def _(v):
            for u in range(4):
                val = plsc.load_gather(sumv, [zi + (v + u)])
                repv.at[pl.ds((v + u) * L, L)][...] = val

        dst_off = pl.ds((sid * RPS) * 16, RPS * 16)

        @pl.when(cid == 0)
        def _():
            pltpu.sync_copy(repv, p0_hbm.at[dst_off])

        @pl.when(cid == 1)
        def _():
            pltpu.sync_copy(repv, p1_hbm.at[dst_off])

    return k


@functools.cache
def _make_sc_aggregate(trows):
    @functools.partial(
        pl.kernel,
        out_type=[_f32(ACC_ROWS, 16), _f32(ACC_ROWS, 16)],
        mesh=_mesh(),
        compiler_params=_SC_PARAMS,
        scratch_types=[
            pltpu.VMEM((NCH, KCH), jnp.int32),
            pltpu.VMEM((NCH, KCH), jnp.int32),
            [pltpu.VMEM((KCH, 16), jnp.float32)] * NBUF,
            pltpu.VMEM_SHARED((Z_SH_ROWS, 16), jnp.float32),
            pltpu.VMEM_SHARED((ACC_ROWS, 16), jnp.float32),
            [pltpu.SemaphoreType.DMA] * NBUF,
            [pltpu.SemaphoreType.DMA] * NBUF,
        ],
    )
    def k(eidx_hbm, z_hbm, zeros_hbm, pad_hbm, p0_hbm, p1_hbm,
          srcv, dstv, bufs, z_sh, acc, gsems, ssems):
        cid = lax.axis_index("c")
        sid = lax.axis_index("s")
        wid = cid * NS + sid
        rs = pl.ds(sid * RPS, RPS)
        _load_edge_chunks(eidx_hbm, 0, wid, srcv, pad_hbm, gsems[0])
        _load_edge_chunks(eidx_hbm, 1, wid, dstv, pad_hbm, gsems[1])
        pltpu.sync_copy(zeros_hbm.at[rs], acc.at[rs])

        if trows == Z_SH_ROWS:
            zs = pl.ds(sid * 640, 640)
            pltpu.sync_copy(z_hbm.at[zs], z_sh.at[zs])
        else:
            @pl.when(sid < NS - 1)
            def _():
                zs = pl.ds(sid * 640, 640)
                pltpu.sync_copy(z_hbm.at[zs], z_sh.at[zs])

            @pl.when(sid == NS - 1)
            def _():
                zs = pl.ds((NS - 1) * 640, trows - (NS - 1) * 640)
                pltpu.sync_copy(z_hbm.at[zs], z_sh.at[zs])

        plsc.subcore_barrier()

        def fire_gather(c):
            return pltpu.async_copy(
                z_sh.at[srcv.at[c]], bufs[c % NBUF], gsems[c % NBUF])

        def fire_scatter(c):
            return pltpu.async_copy(
                bufs[c % NBUF], acc.at[dstv.at[c]], ssems[c % NBUF],
                add=True)

        depth = 2
        gh = {c: fire_gather(c) for c in range(depth)}
        sh = {}
        for c in range(NCH):
            gh[c].wait()
            sh[c] = fire_scatter(c)
            nxt = c + depth
            if nxt < NCH:
                if nxt >= NBUF:
                    sh[nxt - NBUF].wait()
                    del sh[nxt - NBUF]
                gh[nxt] = fire_gather(nxt)
        for c in sorted(sh):
            sh[c].wait()

        plsc.subcore_barrier()

        @pl.when(cid == 0)
        def _():
            pltpu.sync_copy(acc.at[rs], p0_hbm.at[rs])

        @pl.when(cid == 1)
        def _():
            pltpu.sync_copy(acc.at[rs], p1_hbm.at[rs])

    return k


@functools.cache
def _make_sc_score():
    @functools.partial(
        pl.kernel,
        out_type=_f32(EX_PAD),
        mesh=_mesh(),
        compiler_params=_SC_PARAMS,
        scratch_types=[
            pltpu.VMEM((RPS, 16), jnp.float32),
            pltpu.VMEM((RPS, 16), jnp.float32),
            pltpu.VMEM((RPS, 16), jnp.float32),
            pltpu.VMEM((RPS, 16), jnp.float32),
            pltpu.VMEM((RPS,), jnp.float32),
            pltpu.VMEM((RPS,), jnp.float32),
            pltpu.VMEM((16,), jnp.float32),
            pltpu.VMEM_SHARED((ACC_ROWS,), jnp.float32),
            pltpu.VMEM_SHARED((ACC_ROWS,), jnp.float32),
            pltpu.VMEM((N,), jnp.float32),
            pltpu.VMEM((N,), jnp.float32),
            pltpu.VMEM((EX_T,), jnp.int32),
            pltpu.VMEM((EX_T,), jnp.int32),
            pltpu.VMEM((EX_T,), jnp.float32),
        ],
    )
    def k(q0_hbm, q1_hbm, t_hbm, y_hbm, cv_hbm, ex0_hbm, ex1_hbm, out_hbm,
          q0v, q1v, tv, yv, siv, sjv, cvv,
          si_sh, sj_sh, sit, sjt, e0v, e1v, outv):
        cid = lax.axis_index("c")
        sid = lax.axis_index("s")
        wid = cid * NS + sid
        rs = pl.ds(sid * RPS, RPS)
        pltpu.sync_copy(q0_hbm.at[rs], q0v)
        pltpu.sync_copy(q1_hbm.at[rs], q1v)
        pltpu.sync_copy(t_hbm.at[rs], tv)
        pltpu.sync_copy(y_hbm.at[rs], yv)
        pltpu.sync_copy(cv_hbm, cvv)
        pltpu.sync_copy(ex0_hbm.at[wid], e0v)
        pltpu.sync_copy(ex1_hbm.at[wid], e1v)

        iot = lax.iota(jnp.int32, L)
        zer = jnp.zeros((L,), jnp.int32)
        one = zer + 1
        ci = plsc.load_gather(cvv, [zer])
        cj = plsc.load_gather(cvv, [one])

        starts = [i * L for i in range(RPS // L)] + [RPS - L]
        for st in starts:
            ridx = iot + st
            yq = plsc.load_gather(yv, [ridx, zer])
            a0 = plsc.load_gather(q0v, [ridx, zer])
            a1 = plsc.load_gather(q1v, [ridx, zer])
            tt = plsc.load_gather(tv, [ridx, zer])
            siv.at[pl.ds(st, L)][...] = yq * (a0 + a1 + tt) + ci
            b0 = plsc.load_gather(q0v, [ridx, one])
            b1 = plsc.load_gather(q1v, [ridx, one])
            bt = plsc.load_gather(tv, [ridx, one])
            sjv.at[pl.ds(st, L)][...] = yq * (b0 + b1 + bt) + cj

        pltpu.sync_copy(siv, si_sh.at[rs])
        pltpu.sync_copy(sjv, sj_sh.at[rs])
        plsc.subcore_barrier()
        pltpu.sync_copy(si_sh.at[pl.ds(0, N)], sit)
        pltpu.sync_copy(sj_sh.at[pl.ds(0, N)], sjt)

        @pl.loop(0, EX_T, step=L)
        def _(i):
            i0 = e0v.at[pl.ds(i, L)][...]
            i1 = e1v.at[pl.ds(i, L)][...]
            a = plsc.load_gather(sit, [i0])
            b = plsc.load_gather(sjt, [i1])
            outv.at[pl.ds(i, L)][...] = 1.0 / (1.0 + jnp.exp(-(a + b)))

        pltpu.sync_copy(outv, out_hbm.at[pl.ds(wid * EX_T, EX_T)])

    return k


NP1 = N // 8
NPA = ACC_ROWS // 8
NPT = Z_SH_ROWS // 8


def _tc1_body(x_ref, w1_ref, d0_ref, d1_ref, z1_ref, y_ref):
    yp = lax.rsqrt(d0_ref[...] + d1_ref[...] + 1.0)
    xw = jnp.dot(x_ref[...], w1_ref[...], preferred_element_type=jnp.float32,
                 precision=lax.Precision.HIGHEST)
    z1_ref[...] = xw * yp[0:NP1, :]
    y_ref[...] = yp


def _tc1(xp, w1blk, d0p, d1p):
    return pl.pallas_call(
        _tc1_body, out_shape=[_f32(NP1, 128), _f32(NPA, 128)]
    )(xp, w1blk, d0p, d1p)


def _tc2_body(y_ref, z1_ref, p0_ref, p1_ref, b1_ref, w2_ref, wrep_ref, t_ref):
    yp = y_ref[0:NP1, :]
    h1 = jnp.maximum(yp * (p0_ref[0:NP1, :] + p1_ref[0:NP1, :] + z1_ref[...])
                     + b1_ref[...], 0.0)
    xw2 = jnp.dot(h1, w2_ref[...], preferred_element_type=jnp.float32,
                  precision=lax.Precision.HIGHEST)
    t_ref[0:NP1, :] = jnp.dot(xw2, wrep_ref[...],
                              preferred_element_type=jnp.float32,
                              precision=lax.Precision.HIGHEST) * yp
    t_ref[NP1:NPT, :] = jnp.zeros((NPT - NP1, 128), jnp.float32)


def _tc2(yp, z1p, p0p, p1p, b1rep, w2blk, wrepblk):
    return pl.pallas_call(_tc2_body, out_shape=_f32(NPT, 128))(
        yp, z1p, p0p, p1p, b1rep, w2blk, wrepblk)


def kernel(x, edge_index, examples, W1, b1, W2, b2, Wfc, bfc):
    eidx = edge_index.astype(jnp.int32)
    xpad = EX_PAD - NEX
    ex0 = jnp.concatenate(
        [examples[:, 0].astype(jnp.int32), jnp.zeros((xpad,), jnp.int32)]
    ).reshape(NW, EX_T)
    ex1 = jnp.concatenate(
        [examples[:, 1].astype(jnp.int32), jnp.zeros((xpad,), jnp.int32)]
    ).reshape(NW, EX_T)

    zeros16 = jnp.zeros((ACC_ROWS, 16), jnp.float32)
    padv = jnp.full((PAD,), DUMMY, jnp.int32)

    wi = Wfc[:32, 0]
    wj = Wfc[32:, 0]
    wrep = jnp.tile(jnp.stack([wi, wj], axis=1), (1, 8))
    cvec = jnp.tile(
        jnp.stack([jnp.dot(b2, wi) + bfc[0], jnp.dot(b2, wj)]), (8,))
    eye8 = jnp.eye(8, dtype=jnp.float32)
    w1blk = jnp.kron(eye8, W1)
    w2blk = jnp.kron(eye8, W2)
    wrepblk = jnp.kron(eye8, wrep)
    b1rep = jnp.tile(b1.reshape(1, 16), (1, 8))
    xp = x.reshape(NP1, 1024)

    d0, d1 = _make_sc_degree()(eidx)
    z1p, yp = _tc1(xp, w1blk, d0.reshape(NPA, 128), d1.reshape(NPA, 128))
    p0, p1 = _make_sc_aggregate(N)(eidx, z1p.reshape(N, 16), zeros16, padv)
    tp = _tc2(yp, z1p, p0.reshape(NPA, 128), p1.reshape(NPA, 128),
              b1rep, w2blk, wrepblk)
    t16 = tp.reshape(Z_SH_ROWS, 16)
    q0, q1 = _make_sc_aggregate(Z_SH_ROWS)(eidx, t16, zeros16, padv)
    out = _make_sc_score()(q0, q1, t16, yp.reshape(ACC_ROWS, 16),
                           cvec, ex0, ex1)
    return out[:NEX]


# speedup vs baseline: 2.2045x; 1.0271x over previous
import functools

import jax
import jax.numpy as jnp
from jax import lax
from jax.experimental import pallas as pl
from jax.experimental.pallas import tpu as pltpu
from jax.experimental.pallas import tpu_sc as plsc

N = 10000
E = 320000
NEX = 100000
NC, NS, L = 2, 16, 16
NW = NC * NS

EPT = E // NW
KCH = 512
NCH = 20
TAIL = EPT - (NCH - 1) * KCH
PAD = KCH - TAIL
NBUF = 4
Z_SH_ROWS = 10240
DUMMY = N
ACC_ROWS = 10112
RPS = ACC_ROWS // NS

EX_T = 3200
EX_PAD = NW * EX_T


def _f32(*shape):
    return jax.ShapeDtypeStruct(shape, jnp.float32)


@functools.cache
def _mesh():
    return plsc.VectorSubcoreMesh(
        core_axis_name="c", subcore_axis_name="s",
        num_cores=NC, num_subcores=NS)


_SC_PARAMS = pltpu.CompilerParams(
    use_tc_tiling_on_sc=False, needs_layout_passes=False)


def _load_edge_chunks(eidx_hbm, row, wid, idxv, pad_hbm, sem):
    base = wid * EPT
    hs = [pltpu.async_copy(eidx_hbm.at[row, pl.ds(base + c * KCH, KCH)],
                           idxv.at[c, pl.ds(0, KCH)], sem)
          for c in range(NCH - 1)]
    hs.append(pltpu.async_copy(
        eidx_hbm.at[row, pl.ds(base + (NCH - 1) * KCH, TAIL)],
        idxv.at[NCH - 1, pl.ds(0, TAIL)], sem))
    hs.append(pltpu.async_copy(
        pad_hbm, idxv.at[NCH - 1, pl.ds(TAIL, PAD)], sem))
    return hs


@functools.cache
def _make_sc_degree():
    @functools.partial(
        pl.kernel,
        out_type=[_f32(ACC_ROWS * 16), _f32(ACC_ROWS * 16)],
        mesh=_mesh(),
        compiler_params=_SC_PARAMS,
        scratch_types=[
            pltpu.VMEM((EPT,), jnp.int32),
            pltpu.VMEM((ACC_ROWS,), jnp.float32),
            pltpu.VMEM_SHARED((NS, ACC_ROWS), jnp.float32),
            pltpu.VMEM((NS * RPS,), jnp.float32),
            pltpu.VMEM((RPS,), jnp.float32),
            pltpu.VMEM((RPS * 16,), jnp.float32),
            pltpu.SemaphoreType.DMA,
        ],
    )
    def k(eidx_hbm, p0_hbm, p1_hbm, dstv, tbl, slab_sh, slabv, sumv, repv,
          sem):
        cid = lax.axis_index("c")
        sid = lax.axis_index("s")
        wid = cid * NS + sid
        pltpu.sync_copy(eidx_hbm.at[1, pl.ds(wid * EPT, EPT)], dstv)

        zf = jnp.zeros((L,), jnp.float32)
        onef = zf + 1.0

        @pl.loop(0, ACC_ROWS, step=L)
        def _(i):
            tbl.at[pl.ds(i, L)][...] = zf

        @pl.loop(0, EPT, step=L)
        def _(e):
            idx = dstv.at[pl.ds(e, L)][...]
            plsc.addupdate_scatter(tbl, [idx], onef)

        pltpu.sync_copy(tbl, slab_sh.at[sid])
        plsc.subcore_barrier()

        hs = [pltpu.async_copy(slab_sh.at[r, pl.ds(sid * RPS, RPS)],
                               slabv.at[pl.ds(r * RPS, RPS)], sem)
              for r in range(NS)]
        for h in hs:
            h.wait()

        starts = [i * L for i in range(RPS // L)] + [RPS - L]
        for st in starts:
            tot = slabv.at[pl.ds(st, L)][...]
            for r in range(1, NS):
                tot = tot + slabv.at[pl.ds(r * RPS + st, L)][...]
            sumv.at[pl.ds(st, L)][...] = tot

        zi = jnp.zeros((L,), jnp.int32)

        @pl.loop(0, RPS, step=4)
        def _(v):
            for u in range(4):
                val = plsc.load_gather(sumv, [zi + (v + u)])
                repv.at[pl.ds((v + u) * L, L)][...] = val

        dst_off = pl.ds((sid * RPS) * 16, RPS * 16)

        @pl.when(cid == 0)
        def _():
            pltpu.sync_copy(repv, p0_hbm.at[dst_off])

        @pl.when(cid == 1)
        def _():
            pltpu.sync_copy(repv, p1_hbm.at[dst_off])

    return k


@functools.cache
def _make_sc_aggregate():
    @functools.partial(
        pl.kernel,
        out_type=[_f32(ACC_ROWS, 16), _f32(ACC_ROWS, 16)],
        mesh=_mesh(),
        compiler_params=_SC_PARAMS,
        scratch_types=[
            pltpu.VMEM((NCH, KCH), jnp.int32),
            pltpu.VMEM((NCH, KCH), jnp.int32),
            [pltpu.VMEM((KCH, 16), jnp.float32)] * NBUF,
            pltpu.VMEM_SHARED((Z_SH_ROWS, 16), jnp.float32),
            pltpu.VMEM_SHARED((ACC_ROWS, 16), jnp.float32),
            [pltpu.SemaphoreType.DMA] * NBUF,
            [pltpu.SemaphoreType.DMA] * NBUF,
        ],
    )
    def k(eidx_hbm, z_hbm, zeros_hbm, pad_hbm, p0_hbm, p1_hbm,
          srcv, dstv, bufs, z_sh, acc, gsems, ssems):
        cid = lax.axis_index("c")
        sid = lax.axis_index("s")
        wid = cid * NS + sid
        rs = pl.ds(sid * RPS, RPS)
        hs = _load_edge_chunks(eidx_hbm, 0, wid, srcv, pad_hbm, gsems[0])
        hs += _load_edge_chunks(eidx_hbm, 1, wid, dstv, pad_hbm, gsems[1])
        hs.append(pltpu.async_copy(zeros_hbm.at[rs], acc.at[rs], gsems[2]))

        zs = pl.ds(sid * 640, 640)
        hs.append(pltpu.async_copy(z_hbm.at[zs], z_sh.at[zs], gsems[3]))
        for h in hs:
            h.wait()

        plsc.subcore_barrier()

        def fire_gather(c):
            return pltpu.async_copy(
                z_sh.at[srcv.at[c]], bufs[c % NBUF], gsems[c % NBUF])

        def fire_scatter(c):
            return pltpu.async_copy(
                bufs[c % NBUF], acc.at[dstv.at[c]], ssems[c % NBUF],
                add=True)

        depth = 2
        gh = {c: fire_gather(c) for c in range(depth)}
        sh = {}
        for c in range(NCH):
            gh[c].wait()
            sh[c] = fire_scatter(c)
            nxt = c + depth
            if nxt < NCH:
                if nxt >= NBUF:
                    sh[nxt - NBUF].wait()
                    del sh[nxt - NBUF]
                gh[nxt] = fire_gather(nxt)
        for c in sorted(sh):
            sh[c].wait()

        plsc.subcore_barrier()

        @pl.when(cid == 0)
        def _():
            pltpu.sync_copy(acc.at[rs], p0_hbm.at[rs])

        @pl.when(cid == 1)
        def _():
            pltpu.sync_copy(acc.at[rs], p1_hbm.at[rs])

    return k


@functools.cache
def _make_sc_score():
    @functools.partial(
        pl.kernel,
        out_type=_f32(EX_PAD),
        mesh=_mesh(),
        compiler_params=_SC_PARAMS,
        scratch_types=[
            pltpu.VMEM((RPS, 16), jnp.float32),
            pltpu.VMEM((RPS, 16), jnp.float32),
            pltpu.VMEM((RPS, 16), jnp.float32),
            pltpu.VMEM((RPS, 16), jnp.float32),
            pltpu.VMEM((RPS,), jnp.float32),
            pltpu.VMEM((RPS,), jnp.float32),
            pltpu.VMEM((16,), jnp.float32),
            pltpu.VMEM_SHARED((ACC_ROWS,), jnp.float32),
            pltpu.VMEM_SHARED((ACC_ROWS,), jnp.float32),
            pltpu.VMEM((N,), jnp.float32),
            pltpu.VMEM((N,), jnp.float32),
            pltpu.VMEM((EX_T,), jnp.int32),
            pltpu.VMEM((EX_T,), jnp.int32),
            pltpu.VMEM((EX_T,), jnp.float32),
        ],
    )
    def k(q0_hbm, q1_hbm, t_hbm, y_hbm, cv_hbm, ex0_hbm, ex1_hbm, out_hbm,
          q0v, q1v, tv, yv, siv, sjv, cvv,
          si_sh, sj_sh, sit, sjt, e0v, e1v, outv):
        cid = lax.axis_index("c")
        sid = lax.axis_index("s")
        wid = cid * NS + sid
        rs = pl.ds(sid * RPS, RPS)
        pltpu.sync_copy(q0_hbm.at[rs], q0v)
        pltpu.sync_copy(q1_hbm.at[rs], q1v)
        pltpu.sync_copy(t_hbm.at[rs], tv)
        pltpu.sync_copy(y_hbm.at[rs], yv)
        pltpu.sync_copy(cv_hbm, cvv)
        pltpu.sync_copy(ex0_hbm.at[wid], e0v)
        pltpu.sync_copy(ex1_hbm.at[wid], e1v)

        iot = lax.iota(jnp.int32, L)
        zer = jnp.zeros((L,), jnp.int32)
        one = zer + 1
        ci = plsc.load_gather(cvv, [zer])
        cj = plsc.load_gather(cvv, [one])

        starts = [i * L for i in range(RPS // L)] + [RPS - L]
        for st in starts:
            ridx = iot + st
            yq = plsc.load_gather(yv, [ridx, zer])
            a0 = plsc.load_gather(q0v, [ridx, zer])
            a1 = plsc.load_gather(q1v, [ridx, zer])
            tt = plsc.load_gather(tv, [ridx, zer])
            siv.at[pl.ds(st, L)][...] = yq * (a0 + a1 + tt) + ci
            b0 = plsc.load_gather(q0v, [ridx, one])
            b1 = plsc.load_gather(q1v, [ridx, one])
            bt = plsc.load_gather(tv, [ridx, one])
            sjv.at[pl.ds(st, L)][...] = yq * (b0 + b1 + bt) + cj

        pltpu.sync_copy(siv, si_sh.at[rs])
        pltpu.sync_copy(sjv, sj_sh.at[rs])
        plsc.subcore_barrier()
        pltpu.sync_copy(si_sh.at[pl.ds(0, N)], sit)
        pltpu.sync_copy(sj_sh.at[pl.ds(0, N)], sjt)

        @pl.loop(0, EX_T, step=L)
        def _(i):
            i0 = e0v.at[pl.ds(i, L)][...]
            i1 = e1v.at[pl.ds(i, L)][...]
            a = plsc.load_gather(sit, [i0])
            b = plsc.load_gather(sjt, [i1])
            outv.at[pl.ds(i, L)][...] = 1.0 / (1.0 + jnp.exp(-(a + b)))

        pltpu.sync_copy(outv, out_hbm.at[pl.ds(wid * EX_T, EX_T)])

    return k


NP1 = N // 8
NPA = ACC_ROWS // 8
NPT = Z_SH_ROWS // 8


def _tc1_body(x_ref, w1_ref, d0_ref, d1_ref, z1_ref, y_ref):
    yp = lax.rsqrt(d0_ref[...] + d1_ref[...] + 1.0)
    xw = jnp.dot(x_ref[...], w1_ref[...], preferred_element_type=jnp.float32,
                 precision=lax.Precision.HIGHEST)
    z1_ref[0:NP1, :] = xw * yp[0:NP1, :]
    z1_ref[NP1:NPT, :] = jnp.zeros((NPT - NP1, 128), jnp.float32)
    y_ref[...] = yp


def _tc1(xp, w1blk, d0p, d1p):
    return pl.pallas_call(
        _tc1_body, out_shape=[_f32(NPT, 128), _f32(NPA, 128)]
    )(xp, w1blk, d0p, d1p)


def _tc2_body(y_ref, z1_ref, p0_ref, p1_ref, b1_ref, w2_ref, wrep_ref, t_ref):
    yp = y_ref[0:NP1, :]
    h1 = jnp.maximum(
        yp * (p0_ref[0:NP1, :] + p1_ref[0:NP1, :] + z1_ref[0:NP1, :])
        + b1_ref[...], 0.0)
    xw2 = jnp.dot(h1, w2_ref[...], preferred_element_type=jnp.float32,
                  precision=lax.Precision.HIGHEST)
    t_ref[0:NP1, :] = jnp.dot(xw2, wrep_ref[...],
                              preferred_element_type=jnp.float32,
                              precision=lax.Precision.HIGHEST) * yp
    t_ref[NP1:NPT, :] = jnp.zeros((NPT - NP1, 128), jnp.float32)


def _tc2(yp, z1p, p0p, p1p, b1rep, w2blk, wrepblk):
    return pl.pallas_call(_tc2_body, out_shape=_f32(NPT, 128))(
        yp, z1p, p0p, p1p, b1rep, w2blk, wrepblk)


def kernel(x, edge_index, examples, W1, b1, W2, b2, Wfc, bfc):
    eidx = edge_index.astype(jnp.int32)
    xpad = EX_PAD - NEX
    ex0 = jnp.concatenate(
        [examples[:, 0].astype(jnp.int32), jnp.zeros((xpad,), jnp.int32)]
    ).reshape(NW, EX_T)
    ex1 = jnp.concatenate(
        [examples[:, 1].astype(jnp.int32), jnp.zeros((xpad,), jnp.int32)]
    ).reshape(NW, EX_T)

    zeros16 = jnp.zeros((ACC_ROWS, 16), jnp.float32)
    padv = jnp.full((PAD,), DUMMY, jnp.int32)

    wi = Wfc[:32, 0]
    wj = Wfc[32:, 0]
    wrep = jnp.tile(jnp.stack([wi, wj], axis=1), (1, 8))
    cvec = jnp.tile(
        jnp.stack([jnp.dot(b2, wi) + bfc[0], jnp.dot(b2, wj)]), (8,))
    eye8 = jnp.eye(8, dtype=jnp.float32)
    w1blk = jnp.kron(eye8, W1)
    w2blk = jnp.kron(eye8, W2)
    wrepblk = jnp.kron(eye8, wrep)
    b1rep = jnp.tile(b1.reshape(1, 16), (1, 8))
    xp = x.reshape(NP1, 1024)

    d0, d1 = _make_sc_degree()(eidx)
    z1p, yp = _tc1(xp, w1blk, d0.reshape(NPA, 128), d1.reshape(NPA, 128))
    p0, p1 = _make_sc_aggregate()(eidx, z1p.reshape(Z_SH_ROWS, 16), zeros16, padv)
    tp = _tc2(yp, z1p, p0.reshape(NPA, 128), p1.reshape(NPA, 128),
              b1rep, w2blk, wrepblk)
    t16 = tp.reshape(Z_SH_ROWS, 16)
    q0, q1 = _make_sc_aggregate()(eidx, t16, zeros16, padv)
    out = _make_sc_score()(q0, q1, t16, yp.reshape(ACC_ROWS, 16),
                           cvec, ex0, ex1)
    return out[:NEX]


# speedup vs baseline: 2.2410x; 1.0166x over previous
import functools

import jax
import jax.numpy as jnp
from jax import lax
from jax.experimental import pallas as pl
from jax.experimental.pallas import tpu as pltpu
from jax.experimental.pallas import tpu_sc as plsc

N = 10000
E = 320000
NEX = 100000
NC, NS, L = 2, 16, 16
NW = NC * NS

EPT = E // NW
KCH = 512
NCH = 20
TAIL = EPT - (NCH - 1) * KCH
PAD = KCH - TAIL
NBUF = 6
Z_SH_ROWS = 10240
DUMMY = N
ACC_ROWS = 10112
RPS = ACC_ROWS // NS

EX_T = 3200
EX_PAD = NW * EX_T


def _f32(*shape):
    return jax.ShapeDtypeStruct(shape, jnp.float32)


@functools.cache
def _mesh():
    return plsc.VectorSubcoreMesh(
        core_axis_name="c", subcore_axis_name="s",
        num_cores=NC, num_subcores=NS)


_SC_PARAMS = pltpu.CompilerParams(
    use_tc_tiling_on_sc=False, needs_layout_passes=False)


def _load_edge_chunks(eidx_hbm, row, wid, idxv, pad_hbm, sem):
    base = wid * EPT
    hs = [pltpu.async_copy(eidx_hbm.at[row, pl.ds(base + c * KCH, KCH)],
                           idxv.at[c, pl.ds(0, KCH)], sem)
          for c in range(NCH - 1)]
    hs.append(pltpu.async_copy(
        eidx_hbm.at[row, pl.ds(base + (NCH - 1) * KCH, TAIL)],
        idxv.at[NCH - 1, pl.ds(0, TAIL)], sem))
    hs.append(pltpu.async_copy(
        pad_hbm, idxv.at[NCH - 1, pl.ds(TAIL, PAD)], sem))
    return hs


@functools.cache
def _make_sc_degree():
    @functools.partial(
        pl.kernel,
        out_type=[_f32(ACC_ROWS * 16), _f32(ACC_ROWS * 16)],
        mesh=_mesh(),
        compiler_params=_SC_PARAMS,
        scratch_types=[
            pltpu.VMEM((EPT,), jnp.int32),
            pltpu.VMEM((ACC_ROWS,), jnp.float32),
            pltpu.VMEM_SHARED((NS, ACC_ROWS), jnp.float32),
            pltpu.VMEM((NS * RPS,), jnp.float32),
            pltpu.VMEM((RPS,), jnp.float32),
            pltpu.VMEM((RPS * 16,), jnp.float32),
            pltpu.SemaphoreType.DMA,
        ],
    )
    def k(eidx_hbm, p0_hbm, p1_hbm, dstv, tbl, slab_sh, slabv, sumv, repv,
          sem):
        cid = lax.axis_index("c")
        sid = lax.axis_index("s")
        wid = cid * NS + sid
        pltpu.sync_copy(eidx_hbm.at[1, pl.ds(wid * EPT, EPT)], dstv)

        zf = jnp.zeros((L,), jnp.float32)
        onef = zf + 1.0

        @pl.loop(0, ACC_ROWS, step=L)
        def _(i):
            tbl.at[pl.ds(i, L)][...] = zf

        @pl.loop(0, EPT, step=L)
        def _(e):
            idx = dstv.at[pl.ds(e, L)][...]
            plsc.addupdate_scatter(tbl, [idx], onef)

        pltpu.sync_copy(tbl, slab_sh.at[sid])
        plsc.subcore_barrier()

        hs = [pltpu.async_copy(slab_sh.at[r, pl.ds(sid * RPS, RPS)],
                               slabv.at[pl.ds(r * RPS, RPS)], sem)
              for r in range(NS)]
        for h in hs:
            h.wait()

        starts = [i * L for i in range(RPS // L)] + [RPS - L]
        for st in starts:
            tot = slabv.at[pl.ds(st, L)][...]
            for r in range(1, NS):
                tot = tot + slabv.at[pl.ds(r * RPS + st, L)][...]
            sumv.at[pl.ds(st, L)][...] = tot

        zi = jnp.zeros((L,), jnp.int32)

        @pl.loop(0, RPS, step=4)
        def _(v):
            for u in range(4):
                val = plsc.load_gather(sumv, [zi + (v + u)])
                repv.at[pl.ds((v + u) * L, L)][...] = val

        dst_off = pl.ds((sid * RPS) * 16, RPS * 16)

        @pl.when(cid == 0)
        def _():
            pltpu.sync_copy(repv, p0_hbm.at[dst_off])

        @pl.when(cid == 1)
        def _():
            pltpu.sync_copy(repv, p1_hbm.at[dst_off])

    return k


@functools.cache
def _make_sc_aggregate():
    @functools.partial(
        pl.kernel,
        out_type=[_f32(ACC_ROWS, 16), _f32(ACC_ROWS, 16)],
        mesh=_mesh(),
        compiler_params=_SC_PARAMS,
        scratch_types=[
            pltpu.VMEM((NCH, KCH), jnp.int32),
            pltpu.VMEM((NCH, KCH), jnp.int32),
            [pltpu.VMEM((KCH, 16), jnp.float32)] * NBUF,
            pltpu.VMEM_SHARED((Z_SH_ROWS, 16), jnp.float32),
            pltpu.VMEM_SHARED((ACC_ROWS, 16), jnp.float32),
            [pltpu.SemaphoreType.DMA] * NBUF,
            [pltpu.SemaphoreType.DMA] * NBUF,
        ],
    )
    def k(eidx_hbm, z_hbm, zeros_hbm, pad_hbm, p0_hbm, p1_hbm,
          srcv, dstv, bufs, z_sh, acc, gsems, ssems):
        cid = lax.axis_index("c")
        sid = lax.axis_index("s")
        wid = cid * NS + sid
        rs = pl.ds(sid * RPS, RPS)
        hs = _load_edge_chunks(eidx_hbm, 0, wid, srcv, pad_hbm, gsems[0])
        hs += _load_edge_chunks(eidx_hbm, 1, wid, dstv, pad_hbm, gsems[1])
        hs.append(pltpu.async_copy(zeros_hbm.at[rs], acc.at[rs], gsems[2]))

        zs = pl.ds(sid * 640, 640)
        hs.append(pltpu.async_copy(z_hbm.at[zs], z_sh.at[zs], gsems[3]))
        for h in hs:
            h.wait()

        plsc.subcore_barrier()

        def fire_gather(c):
            return pltpu.async_copy(
                z_sh.at[srcv.at[c]], bufs[c % NBUF], gsems[c % NBUF])

        def fire_scatter(c):
            return pltpu.async_copy(
                bufs[c % NBUF], acc.at[dstv.at[c]], ssems[c % NBUF],
                add=True)

        depth = 3
        gh = {c: fire_gather(c) for c in range(depth)}
        sh = {}
        for c in range(NCH):
            gh[c].wait()
            sh[c] = fire_scatter(c)
            nxt = c + depth
            if nxt < NCH:
                if nxt >= NBUF:
                    sh[nxt - NBUF].wait()
                    del sh[nxt - NBUF]
                gh[nxt] = fire_gather(nxt)
        for c in sorted(sh):
            sh[c].wait()

        plsc.subcore_barrier()

        @pl.when(cid == 0)
        def _():
            pltpu.sync_copy(acc.at[rs], p0_hbm.at[rs])

        @pl.when(cid == 1)
        def _():
            pltpu.sync_copy(acc.at[rs], p1_hbm.at[rs])

    return k


@functools.cache
def _make_sc_score():
    @functools.partial(
        pl.kernel,
        out_type=_f32(NEX),
        mesh=_mesh(),
        compiler_params=_SC_PARAMS,
        scratch_types=[
            pltpu.VMEM((RPS, 16), jnp.float32),
            pltpu.VMEM((RPS, 16), jnp.float32),
            pltpu.VMEM((RPS, 16), jnp.float32),
            pltpu.VMEM((RPS, 16), jnp.float32),
            pltpu.VMEM((RPS,), jnp.float32),
            pltpu.VMEM((RPS,), jnp.float32),
            pltpu.VMEM((16,), jnp.float32),
            pltpu.VMEM_SHARED((ACC_ROWS,), jnp.float32),
            pltpu.VMEM_SHARED((ACC_ROWS,), jnp.float32),
            pltpu.VMEM((N,), jnp.float32),
            pltpu.VMEM((N,), jnp.float32),
            pltpu.VMEM((EX_T,), jnp.int32),
            pltpu.VMEM((EX_T,), jnp.int32),
            pltpu.VMEM((EX_T,), jnp.float32),
        ],
    )
    def k(q0_hbm, q1_hbm, t_hbm, y_hbm, cv_hbm, ex0_hbm, ex1_hbm, out_hbm,
          q0v, q1v, tv, yv, siv, sjv, cvv,
          si_sh, sj_sh, sit, sjt, e0v, e1v, outv):
        cid = lax.axis_index("c")
        sid = lax.axis_index("s")
        wid = cid * NS + sid
        rs = pl.ds(sid * RPS, RPS)
        pltpu.sync_copy(q0_hbm.at[rs], q0v)
        pltpu.sync_copy(q1_hbm.at[rs], q1v)
        pltpu.sync_copy(t_hbm.at[rs], tv)
        pltpu.sync_copy(y_hbm.at[rs], yv)
        pltpu.sync_copy(cv_hbm, cvv)
        pltpu.sync_copy(ex0_hbm.at[wid], e0v)
        pltpu.sync_copy(ex1_hbm.at[wid], e1v)

        iot = lax.iota(jnp.int32, L)
        zer = jnp.zeros((L,), jnp.int32)
        one = zer + 1
        ci = plsc.load_gather(cvv, [zer])
        cj = plsc.load_gather(cvv, [one])

        starts = [i * L for i in range(RPS // L)] + [RPS - L]
        for st in starts:
            ridx = iot + st
            yq = plsc.load_gather(yv, [ridx, zer])
            a0 = plsc.load_gather(q0v, [ridx, zer])
            a1 = plsc.load_gather(q1v, [ridx, zer])
            tt = plsc.load_gather(tv, [ridx, zer])
            siv.at[pl.ds(st, L)][...] = yq * (a0 + a1 + tt) + ci
            b0 = plsc.load_gather(q0v, [ridx, one])
            b1 = plsc.load_gather(q1v, [ridx, one])
            bt = plsc.load_gather(tv, [ridx, one])
            sjv.at[pl.ds(st, L)][...] = yq * (b0 + b1 + bt) + cj

        pltpu.sync_copy(siv, si_sh.at[rs])
        pltpu.sync_copy(sjv, sj_sh.at[rs])
        plsc.subcore_barrier()
        pltpu.sync_copy(si_sh.at[pl.ds(0, N)], sit)
        pltpu.sync_copy(sj_sh.at[pl.ds(0, N)], sjt)

        @pl.loop(0, EX_T, step=L)
        def _(i):
            i0 = e0v.at[pl.ds(i, L)][...]
            i1 = e1v.at[pl.ds(i, L)][...]
            a = plsc.load_gather(sit, [i0])
            b = plsc.load_gather(sjt, [i1])
            outv.at[pl.ds(i, L)][...] = 1.0 / (1.0 + jnp.exp(-(a + b)))

        @pl.when(wid < NW - 1)
        def _():
            pltpu.sync_copy(outv, out_hbm.at[pl.ds(wid * EX_T, EX_T)])

        @pl.when(wid == NW - 1)
        def _():
            pltpu.sync_copy(outv.at[pl.ds(0, NEX - (NW - 1) * EX_T)],
                            out_hbm.at[pl.ds((NW - 1) * EX_T,
                                             NEX - (NW - 1) * EX_T)])

    return k


NP1 = N // 8
NPA = ACC_ROWS // 8
NPT = Z_SH_ROWS // 8


def _tc1_body(x_ref, w1_ref, d0_ref, d1_ref, z1_ref, y_ref):
    yp = lax.rsqrt(d0_ref[...] + d1_ref[...] + 1.0)
    xw = jnp.dot(x_ref[...], w1_ref[...], preferred_element_type=jnp.float32,
                 precision=lax.Precision.HIGHEST)
    z1_ref[0:NP1, :] = xw * yp[0:NP1, :]
    z1_ref[NP1:NPT, :] = jnp.zeros((NPT - NP1, 128), jnp.float32)
    y_ref[...] = yp


def _tc1(xp, w1blk, d0p, d1p):
    return pl.pallas_call(
        _tc1_body, out_shape=[_f32(NPT, 128), _f32(NPA, 128)]
    )(xp, w1blk, d0p, d1p)


def _tc2_body(y_ref, z1_ref, p0_ref, p1_ref, b1_ref, w2_ref, wrep_ref, t_ref):
    yp = y_ref[0:NP1, :]
    h1 = jnp.maximum(
        yp * (p0_ref[0:NP1, :] + p1_ref[0:NP1, :] + z1_ref[0:NP1, :])
        + b1_ref[...], 0.0)
    xw2 = jnp.dot(h1, w2_ref[...], preferred_element_type=jnp.float32,
                  precision=lax.Precision.HIGHEST)
    t_ref[0:NP1, :] = jnp.dot(xw2, wrep_ref[...],
                              preferred_element_type=jnp.float32,
                              precision=lax.Precision.HIGHEST) * yp
    t_ref[NP1:NPT, :] = jnp.zeros((NPT - NP1, 128), jnp.float32)


def _tc2(yp, z1p, p0p, p1p, b1rep, w2blk, wrepblk):
    return pl.pallas_call(_tc2_body, out_shape=_f32(NPT, 128))(
        yp, z1p, p0p, p1p, b1rep, w2blk, wrepblk)


def kernel(x, edge_index, examples, W1, b1, W2, b2, Wfc, bfc):
    eidx = edge_index.astype(jnp.int32)
    xpad = EX_PAD - NEX
    ex0 = jnp.concatenate(
        [examples[:, 0].astype(jnp.int32), jnp.zeros((xpad,), jnp.int32)]
    ).reshape(NW, EX_T)
    ex1 = jnp.concatenate(
        [examples[:, 1].astype(jnp.int32), jnp.zeros((xpad,), jnp.int32)]
    ).reshape(NW, EX_T)

    zeros16 = jnp.zeros((ACC_ROWS, 16), jnp.float32)
    padv = jnp.full((PAD,), DUMMY, jnp.int32)

    wi = Wfc[:32, 0]
    wj = Wfc[32:, 0]
    wrep = jnp.tile(jnp.stack([wi, wj], axis=1), (1, 8))
    cvec = jnp.tile(
        jnp.stack([jnp.dot(b2, wi) + bfc[0], jnp.dot(b2, wj)]), (8,))
    eye8 = jnp.eye(8, dtype=jnp.float32)
    w1blk = jnp.kron(eye8, W1)
    w2blk = jnp.kron(eye8, W2)
    wrepblk = jnp.kron(eye8, wrep)
    b1rep = jnp.tile(b1.reshape(1, 16), (1, 8))
    xp = x.reshape(NP1, 1024)

    d0, d1 = _make_sc_degree()(eidx)
    dep = (d0[0] * 0.0).astype(jnp.int32)
    ex0 = ex0 + dep
    ex1 = ex1 + dep
    z1p, yp = _tc1(xp, w1blk, d0.reshape(NPA, 128), d1.reshape(NPA, 128))
    p0, p1 = _make_sc_aggregate()(eidx, z1p.reshape(Z_SH_ROWS, 16), zeros16, padv)
    tp = _tc2(yp, z1p, p0.reshape(NPA, 128), p1.reshape(NPA, 128),
              b1rep, w2blk, wrepblk)
    t16 = tp.reshape(Z_SH_ROWS, 16)
    q0, q1 = _make_sc_aggregate()(eidx, t16, zeros16, padv)
    return _make_sc_score()(q0, q1, t16, yp.reshape(ACC_ROWS, 16),
                            cvec, ex0, ex1)


# speedup vs baseline: 2.3101x; 1.0308x over previous
import functools

import jax
import jax.numpy as jnp
from jax import lax
from jax.experimental import pallas as pl
from jax.experimental.pallas import tpu as pltpu
from jax.experimental.pallas import tpu_sc as plsc

N = 10000
E = 320000
NEX = 100000
NC, NS, L = 2, 16, 16
NW = NC * NS

EPT = E // NW
KCH = 512
NCH = 20
TAIL = EPT - (NCH - 1) * KCH
PAD = KCH - TAIL
NBUF = 6
Z_SH_ROWS = 10240
DUMMY = N
ACC_ROWS = 10112
RPS = ACC_ROWS // NS

EX_T = 3200
EX_PAD = NW * EX_T


def _f32(*shape):
    return jax.ShapeDtypeStruct(shape, jnp.float32)


@functools.cache
def _mesh():
    return plsc.VectorSubcoreMesh(
        core_axis_name="c", subcore_axis_name="s",
        num_cores=NC, num_subcores=NS)


_SC_PARAMS = pltpu.CompilerParams(
    use_tc_tiling_on_sc=False, needs_layout_passes=False)


def _load_edge_chunks(eidx_hbm, row, wid, idxv, pad_hbm, sem):
    base = wid * EPT
    hs = [pltpu.async_copy(eidx_hbm.at[row, pl.ds(base + c * KCH, KCH)],
                           idxv.at[c, pl.ds(0, KCH)], sem)
          for c in range(NCH - 1)]
    hs.append(pltpu.async_copy(
        eidx_hbm.at[row, pl.ds(base + (NCH - 1) * KCH, TAIL)],
        idxv.at[NCH - 1, pl.ds(0, TAIL)], sem))
    hs.append(pltpu.async_copy(
        pad_hbm, idxv.at[NCH - 1, pl.ds(TAIL, PAD)], sem))
    return hs


@functools.cache
def _make_sc_degree():
    @functools.partial(
        pl.kernel,
        out_type=[_f32(ACC_ROWS * 16), _f32(ACC_ROWS * 16)],
        mesh=_mesh(),
        compiler_params=_SC_PARAMS,
        scratch_types=[
            pltpu.VMEM((EPT,), jnp.int32),
            pltpu.VMEM((ACC_ROWS,), jnp.float32),
            pltpu.VMEM_SHARED((NS, ACC_ROWS), jnp.float32),
            pltpu.VMEM((NS * RPS,), jnp.float32),
            pltpu.VMEM((RPS,), jnp.float32),
            pltpu.VMEM((RPS * 16,), jnp.float32),
            pltpu.SemaphoreType.DMA,
        ],
    )
    def k(eidx_hbm, p0_hbm, p1_hbm, dstv, tbl, slab_sh, slabv, sumv, repv,
          sem):
        cid = lax.axis_index("c")
        sid = lax.axis_index("s")
        wid = cid * NS + sid
        pltpu.sync_copy(eidx_hbm.at[1, pl.ds(wid * EPT, EPT)], dstv)

        zf = jnp.zeros((L,), jnp.float32)
        onef = zf + 1.0

        @pl.loop(0, ACC_ROWS, step=L)
        def _(i):
            tbl.at[pl.ds(i, L)][...] = zf

        @pl.loop(0, EPT, step=L)
        def _(e):
            idx = dstv.at[pl.ds(e, L)][...]
            plsc.addupdate_scatter(tbl, [idx], onef)

        pltpu.sync_copy(tbl, slab_sh.at[sid])
        plsc.subcore_barrier()

        hs = [pltpu.async_copy(slab_sh.at[r, pl.ds(sid * RPS, RPS)],
                               slabv.at[pl.ds(r * RPS, RPS)], sem)
              for r in range(NS)]
        for h in hs:
            h.wait()

        starts = [i * L for i in range(RPS // L)] + [RPS - L]
        for st in starts:
            tot = slabv.at[pl.ds(st, L)][...]
            for r in range(1, NS):
                tot = tot + slabv.at[pl.ds(r * RPS + st, L)][...]
            sumv.at[pl.ds(st, L)][...] = tot

        zi = jnp.zeros((L,), jnp.int32)

        @pl.loop(0, RPS, step=4)
        def _(v):
            for u in range(4):
                val = plsc.load_gather(sumv, [zi + (v + u)])
                repv.at[pl.ds((v + u) * L, L)][...] = val

        dst_off = pl.ds((sid * RPS) * 16, RPS * 16)

        @pl.when(cid == 0)
        def _():
            pltpu.sync_copy(repv, p0_hbm.at[dst_off])

        @pl.when(cid == 1)
        def _():
            pltpu.sync_copy(repv, p1_hbm.at[dst_off])

    return k


@functools.cache
def _make_sc_aggregate():
    @functools.partial(
        pl.kernel,
        out_type=[_f32(ACC_ROWS, 16), _f32(ACC_ROWS, 16)],
        mesh=_mesh(),
        compiler_params=_SC_PARAMS,
        scratch_types=[
            pltpu.VMEM((NCH, KCH), jnp.int32),
            pltpu.VMEM((NCH, KCH), jnp.int32),
            [pltpu.VMEM((KCH, 16), jnp.float32)] * NBUF,
            pltpu.VMEM_SHARED((Z_SH_ROWS, 16), jnp.float32),
            pltpu.VMEM_SHARED((ACC_ROWS, 16), jnp.float32),
            [pltpu.SemaphoreType.DMA] * NBUF,
            [pltpu.SemaphoreType.DMA] * NBUF,
        ],
    )
    def k(eidx_hbm, z_hbm, zeros_hbm, pad_hbm, p0_hbm, p1_hbm,
          srcv, dstv, bufs, z_sh, acc, gsems, ssems):
        cid = lax.axis_index("c")
        sid = lax.axis_index("s")
        wid = cid * NS + sid
        rs = pl.ds(sid * RPS, RPS)
        hs = _load_edge_chunks(eidx_hbm, 0, wid, srcv, pad_hbm, gsems[0])
        hs += _load_edge_chunks(eidx_hbm, 1, wid, dstv, pad_hbm, gsems[1])
        hs.append(pltpu.async_copy(zeros_hbm.at[rs], acc.at[rs], gsems[2]))

        zs = pl.ds(sid * 640, 640)
        hs.append(pltpu.async_copy(z_hbm.at[zs], z_sh.at[zs], gsems[3]))
        for h in hs:
            h.wait()

        plsc.subcore_barrier()

        def fire_gather(c):
            return pltpu.async_copy(
                z_sh.at[srcv.at[c]], bufs[c % NBUF], gsems[c % NBUF])

        def fire_scatter(c):
            return pltpu.async_copy(
                bufs[c % NBUF], acc.at[dstv.at[c]], ssems[c % NBUF],
                add=True)

        depth = 3
        gh = {c: fire_gather(c) for c in range(depth)}
        sh = {}
        for c in range(NCH):
            gh[c].wait()
            sh[c] = fire_scatter(c)
            nxt = c + depth
            if nxt < NCH:
                if nxt >= NBUF:
                    sh[nxt - NBUF].wait()
                    del sh[nxt - NBUF]
                gh[nxt] = fire_gather(nxt)
        for c in sorted(sh):
            sh[c].wait()

        plsc.subcore_barrier()

        @pl.when(cid == 0)
        def _():
            pltpu.sync_copy(acc.at[rs], p0_hbm.at[rs])

        @pl.when(cid == 1)
        def _():
            pltpu.sync_copy(acc.at[rs], p1_hbm.at[rs])

    return k


@functools.cache
def _make_sc_score():
    @functools.partial(
        pl.kernel,
        out_type=_f32(NEX),
        mesh=_mesh(),
        compiler_params=_SC_PARAMS,
        scratch_types=[
            pltpu.VMEM((RPS, 16), jnp.float32),
            pltpu.VMEM((RPS, 16), jnp.float32),
            pltpu.VMEM((RPS, 16), jnp.float32),
            pltpu.VMEM((RPS, 16), jnp.float32),
            pltpu.VMEM((RPS,), jnp.float32),
            pltpu.VMEM((RPS,), jnp.float32),
            pltpu.VMEM((16,), jnp.float32),
            pltpu.VMEM_SHARED((ACC_ROWS,), jnp.float32),
            pltpu.VMEM_SHARED((ACC_ROWS,), jnp.float32),
            pltpu.VMEM((N,), jnp.float32),
            pltpu.VMEM((N,), jnp.float32),
            pltpu.VMEM((EX_T,), jnp.int32),
            pltpu.VMEM((EX_T,), jnp.int32),
            pltpu.VMEM((EX_T,), jnp.float32),
            [pltpu.SemaphoreType.DMA] * 4,
        ],
    )
    def k(q0_hbm, q1_hbm, t_hbm, y_hbm, cv_hbm, ex0_hbm, ex1_hbm, out_hbm,
          q0v, q1v, tv, yv, siv, sjv, cvv,
          si_sh, sj_sh, sit, sjt, e0v, e1v, outv, sems):
        cid = lax.axis_index("c")
        sid = lax.axis_index("s")
        wid = cid * NS + sid
        rs = pl.ds(sid * RPS, RPS)
        hs = [pltpu.async_copy(q0_hbm.at[rs], q0v, sems[0]),
              pltpu.async_copy(q1_hbm.at[rs], q1v, sems[1]),
              pltpu.async_copy(t_hbm.at[rs], tv, sems[2]),
              pltpu.async_copy(y_hbm.at[rs], yv, sems[3]),
              pltpu.async_copy(cv_hbm, cvv, sems[0]),
              pltpu.async_copy(ex0_hbm.at[wid], e0v, sems[1]),
              pltpu.async_copy(ex1_hbm.at[wid], e1v, sems[2])]
        for h in hs:
            h.wait()

        iot = lax.iota(jnp.int32, L)
        zer = jnp.zeros((L,), jnp.int32)
        one = zer + 1
        ci = plsc.load_gather(cvv, [zer])
        cj = plsc.load_gather(cvv, [one])

        starts = [i * L for i in range(RPS // L)] + [RPS - L]
        for st in starts:
            ridx = iot + st
            yq = plsc.load_gather(yv, [ridx, zer])
            a0 = plsc.load_gather(q0v, [ridx, zer])
            a1 = plsc.load_gather(q1v, [ridx, zer])
            tt = plsc.load_gather(tv, [ridx, zer])
            siv.at[pl.ds(st, L)][...] = yq * (a0 + a1 + tt) + ci
            b0 = plsc.load_gather(q0v, [ridx, one])
            b1 = plsc.load_gather(q1v, [ridx, one])
            bt = plsc.load_gather(tv, [ridx, one])
            sjv.at[pl.ds(st, L)][...] = yq * (b0 + b1 + bt) + cj

        h1 = pltpu.async_copy(siv, si_sh.at[rs], sems[0])
        h2 = pltpu.async_copy(sjv, sj_sh.at[rs], sems[1])
        h1.wait()
        h2.wait()
        plsc.subcore_barrier()
        h1 = pltpu.async_copy(si_sh.at[pl.ds(0, N)], sit, sems[0])
        h2 = pltpu.async_copy(sj_sh.at[pl.ds(0, N)], sjt, sems[1])
        h1.wait()
        h2.wait()

        @pl.loop(0, EX_T, step=L)
        def _(i):
            i0 = e0v.at[pl.ds(i, L)][...]
            i1 = e1v.at[pl.ds(i, L)][...]
            a = plsc.load_gather(sit, [i0])
            b = plsc.load_gather(sjt, [i1])
            outv.at[pl.ds(i, L)][...] = 1.0 / (1.0 + jnp.exp(-(a + b)))

        @pl.when(wid < NW - 1)
        def _():
            pltpu.sync_copy(outv, out_hbm.at[pl.ds(wid * EX_T, EX_T)])

        @pl.when(wid == NW - 1)
        def _():
            pltpu.sync_copy(outv.at[pl.ds(0, NEX - (NW - 1) * EX_T)],
                            out_hbm.at[pl.ds((NW - 1) * EX_T,
                                             NEX - (NW - 1) * EX_T)])

    return k


NP1 = N // 8
NPA = ACC_ROWS // 8
NPT = Z_SH_ROWS // 8


def _tc1_body(x_ref, w1_ref, d0_ref, d1_ref, z1_ref, y_ref):
    yp = lax.rsqrt(d0_ref[...] + d1_ref[...] + 1.0)
    xw = jnp.dot(x_ref[...], w1_ref[...], preferred_element_type=jnp.float32,
                 precision=lax.Precision.HIGHEST)
    z1_ref[0:NP1, :] = xw * yp[0:NP1, :]
    z1_ref[NP1:NPT, :] = jnp.zeros((NPT - NP1, 128), jnp.float32)
    y_ref[...] = yp


def _tc1(xp, w1blk, d0p, d1p):
    return pl.pallas_call(
        _tc1_body, out_shape=[_f32(NPT, 128), _f32(NPA, 128)]
    )(xp, w1blk, d0p, d1p)


def _tc2_body(y_ref, z1_ref, p0_ref, p1_ref, b1_ref, w2_ref, wrep_ref, t_ref):
    yp = y_ref[0:NP1, :]
    h1 = jnp.maximum(
        yp * (p0_ref[0:NP1, :] + p1_ref[0:NP1, :] + z1_ref[0:NP1, :])
        + b1_ref[...], 0.0)
    xw2 = jnp.dot(h1, w2_ref[...], preferred_element_type=jnp.float32,
                  precision=lax.Precision.HIGHEST)
    t_ref[0:NP1, :] = jnp.dot(xw2, wrep_ref[...],
                              preferred_element_type=jnp.float32,
                              precision=lax.Precision.HIGHEST) * yp
    t_ref[NP1:NPT, :] = jnp.zeros((NPT - NP1, 128), jnp.float32)


def _tc2(yp, z1p, p0p, p1p, b1rep, w2blk, wrepblk):
    return pl.pallas_call(_tc2_body, out_shape=_f32(NPT, 128))(
        yp, z1p, p0p, p1p, b1rep, w2blk, wrepblk)


def kernel(x, edge_index, examples, W1, b1, W2, b2, Wfc, bfc):
    eidx = edge_index.astype(jnp.int32)
    xpad = EX_PAD - NEX
    ex0 = jnp.concatenate(
        [examples[:, 0].astype(jnp.int32), jnp.zeros((xpad,), jnp.int32)]
    ).reshape(NW, EX_T)
    ex1 = jnp.concatenate(
        [examples[:, 1].astype(jnp.int32), jnp.zeros((xpad,), jnp.int32)]
    ).reshape(NW, EX_T)

    zeros16 = jnp.zeros((ACC_ROWS, 16), jnp.float32)
    padv = jnp.full((PAD,), DUMMY, jnp.int32)

    wi = Wfc[:32, 0]
    wj = Wfc[32:, 0]
    wrep = jnp.tile(jnp.stack([wi, wj], axis=1), (1, 8))
    cvec = jnp.tile(
        jnp.stack([jnp.dot(b2, wi) + bfc[0], jnp.dot(b2, wj)]), (8,))
    eye8 = jnp.eye(8, dtype=jnp.float32)
    w1blk = jnp.kron(eye8, W1)
    w2blk = jnp.kron(eye8, W2)
    wrepblk = jnp.kron(eye8, wrep)
    b1rep = jnp.tile(b1.reshape(1, 16), (1, 8))
    xp = x.reshape(NP1, 1024)

    d0, d1 = _make_sc_degree()(eidx)
    dep = (d0[0] * 0.0).astype(jnp.int32)
    ex0 = ex0 + dep
    ex1 = ex1 + dep
    z1p, yp = _tc1(xp, w1blk, d0.reshape(NPA, 128), d1.reshape(NPA, 128))
    p0, p1 = _make_sc_aggregate()(eidx, z1p.reshape(Z_SH_ROWS, 16), zeros16, padv)
    tp = _tc2(yp, z1p, p0.reshape(NPA, 128), p1.reshape(NPA, 128),
              b1rep, w2blk, wrepblk)
    t16 = tp.reshape(Z_SH_ROWS, 16)
    q0, q1 = _make_sc_aggregate()(eidx, t16, zeros16, padv)
    return _make_sc_score()(q0, q1, t16, yp.reshape(ACC_ROWS, 16),
                            cvec, ex0, ex1)
